# baseline (device time: 63679 ns/iter reference)
import jax
import jax.numpy as jnp
from jax import lax
from jax.experimental import pallas as pl
from jax.experimental.pallas import tpu as pltpu

N_DEV = 4
KC = 512


def _kernel4(x, w_mat, variant, kc):
    m_per, k = x.shape
    _, n = w_mat.shape
    n_per = n // N_DEV
    kh = k // 2
    nsteps = kh // kc

    def body(x1_ref, x2_ref, w1_ref, w2_ref, out_ref, acc_ref):
        step = pl.program_id(0)

        if variant == "stream4":
            acc_ref[:kc, :] = w1_ref[...]
            acc_ref[kc:2 * kc, :] = w2_ref[...]
            acc_ref[:m_per, :kc] += x1_ref[...]
            acc_ref[:m_per, kc:2 * kc] += x2_ref[...]

            @pl.when(step == nsteps - 1)
            def _stream_out():
                out_ref[...] = jnp.concatenate(
                    [acc_ref[:m_per, :n_per]] * N_DEV, axis=0
                )
            return

        partial = jnp.dot(
            x1_ref[...].astype(jnp.bfloat16),
            w1_ref[...].astype(jnp.bfloat16),
            preferred_element_type=jnp.float32,
        ) + jnp.dot(
            x2_ref[...].astype(jnp.bfloat16),
            w2_ref[...].astype(jnp.bfloat16),
            preferred_element_type=jnp.float32,
        )

        @pl.when(step == 0)
        def _init_acc():
            acc_ref[...] = partial

        @pl.when(step != 0)
        def _accum():
            acc_ref[...] += partial

        @pl.when(step == nsteps - 1)
        def _raw_out():
            for blk in range(N_DEV):
                out_ref[pl.ds(blk * m_per, m_per), :] = (
                    acc_ref[:, pl.ds(blk * n_per, n_per)]
                )

    return pl.pallas_call(
        body,
        grid=(nsteps,),
        out_shape=jax.ShapeDtypeStruct((N_DEV * m_per, n_per), jnp.float32),
        in_specs=[
            pl.BlockSpec((m_per, kc), lambda s: (0, s)),
            pl.BlockSpec((m_per, kc), lambda s: (0, s + nsteps)),
            pl.BlockSpec((kc, n), lambda s: (s, 0)),
            pl.BlockSpec((kc, n), lambda s: (s + nsteps, 0)),
        ],
        out_specs=pl.BlockSpec((N_DEV * m_per, n_per), lambda s: (0, 0)),
        scratch_shapes=[
            pltpu.VMEM((m_per, n), jnp.float32),
        ],
        compiler_params=pltpu.CompilerParams(
            dimension_semantics=("arbitrary",),
            vmem_limit_bytes=64 * 1024 * 1024,
        ),
    )(x, x, w_mat, w_mat)


def _kernel_v3(x, w_mat, variant, vmem_mb=64):
    m_per, k = x.shape
    _, n = w_mat.shape
    n_per = n // N_DEV
    kh = k // 2

    def body(x1_ref, x2_ref, w1_ref, w2_ref, out_ref,
             xb1_ref, xb2_ref, y_ref, amax_sm, amax_tx_ref, amax_rx_ref,
             q_tx_ref, q_rx_ref, amax_send_sems, amax_recv_sems,
             data_send_sems, data_recv_sems):
        s = pl.program_id(0)
        my = lax.axis_index("i")

        @pl.when(s == 0)
        def _prologue():
            if variant != "v3mm":
                barrier = pltpu.get_barrier_semaphore()
                for d in range(1, N_DEV):
                    pl.semaphore_signal(
                        barrier, inc=1,
                        device_id=((my + d) % N_DEV,),
                        device_id_type=pl.DeviceIdType.MESH,
                    )
                pl.semaphore_wait(barrier, N_DEV - 1)
            xb1_ref[...] = x1_ref[...].astype(jnp.bfloat16)
            xb2_ref[...] = x2_ref[...].astype(jnp.bfloat16)

        partial = jnp.dot(
            xb1_ref[...], w1_ref[...].astype(jnp.bfloat16),
            preferred_element_type=jnp.float32,
        ) + jnp.dot(
            xb2_ref[...], w2_ref[...].astype(jnp.bfloat16),
            preferred_element_type=jnp.float32,
        )
        y_ref[:, pl.ds(s * n_per, n_per)] = partial
        pmax = jnp.max(jnp.abs(partial))

        @pl.when(s == 0)
        def _amax_init():
            amax_sm[0] = pmax

        @pl.when(s != 0)
        def _amax_acc():
            amax_sm[0] = jnp.maximum(amax_sm[0], pmax)

        @pl.when(s == N_DEV - 1)
        def _epilogue():
            local_amax = amax_sm[0]
            if variant in ("v3mm", "v3sync"):
                scale0 = local_amax * (1.0 / 127.0)
                inv0 = 127.0 / local_amax
                for blk in range(N_DEV):
                    q = jnp.clip(
                        jnp.round(
                            y_ref[:, pl.ds(blk * n_per, n_per)] * inv0
                        ), -127.0, 127.0,
                    )
                    out_ref[pl.ds(blk * m_per, m_per), :] = (
                        q * scale0
                    ).astype(jnp.bfloat16)
                return
            amax_tx_ref[...] = jnp.full((8, 128), local_amax, jnp.float32)
            amax_rdmas = []
            for d in range(1, N_DEV):
                rdma = pltpu.make_async_remote_copy(
                    src_ref=amax_tx_ref,
                    dst_ref=amax_rx_ref.at[d - 1],
                    send_sem=amax_send_sems.at[d - 1],
                    recv_sem=amax_recv_sems.at[d - 1],
                    device_id=((my + d) % N_DEV,),
                    device_id_type=pl.DeviceIdType.MESH,
                )
                rdma.start()
                amax_rdmas.append(rdma)
            for rdma in amax_rdmas:
                rdma.wait()
            g_amax = local_amax
            for d in range(1, N_DEV):
                g_amax = jnp.maximum(g_amax, amax_rx_ref[d - 1, 0, 0])
            scale = g_amax * (1.0 / 127.0)
            inv_scale = 127.0 / g_amax

            data_rdmas = []
            for d in range(1, N_DEV):
                tgt = (my + d) % N_DEV
                blk = y_ref[:, pl.ds(tgt * n_per, n_per)]
                q_tx_ref[d - 1] = jnp.clip(
                    jnp.round(blk * inv_scale), -127.0, 127.0
                ).astype(jnp.int8)
                if variant != "v3noa2a":
                    rdma = pltpu.make_async_remote_copy(
                        src_ref=q_tx_ref.at[d - 1],
                        dst_ref=q_rx_ref.at[d - 1],
                        send_sem=data_send_sems.at[d - 1],
                        recv_sem=data_recv_sems.at[d - 1],
                        device_id=(tgt,),
                        device_id_type=pl.DeviceIdType.MESH,
                    )
                    rdma.start()
                    data_rdmas.append(rdma)

            own = y_ref[:, pl.ds(my * n_per, n_per)]
            q_own = jnp.clip(jnp.round(own * inv_scale), -127.0, 127.0)
            out_ref[pl.ds(my * m_per, m_per), :] = (
                q_own * scale
            ).astype(jnp.bfloat16)

            for d in range(1, N_DEV):
                src = (my - d) % N_DEV
                if variant != "v3noa2a":
                    data_rdmas[d - 1].wait()
                    q_src = q_rx_ref[d - 1]
                else:
                    q_src = q_tx_ref[d - 1]
                out_ref[pl.ds(src * m_per, m_per), :] = (
                    q_src.astype(jnp.float32) * scale
                ).astype(jnp.bfloat16)

    return pl.pallas_call(
        body,
        grid=(N_DEV,),
        out_shape=jax.ShapeDtypeStruct((N_DEV * m_per, n_per), jnp.bfloat16),
        in_specs=[
            pl.BlockSpec((m_per, kh), lambda s: (0, 0)),
            pl.BlockSpec((m_per, kh), lambda s: (0, 1)),
            pl.BlockSpec((kh, n_per), lambda s: (0, s)),
            pl.BlockSpec((kh, n_per), lambda s: (1, s)),
        ],
        out_specs=pl.BlockSpec((N_DEV * m_per, n_per), lambda s: (0, 0)),
        scratch_shapes=[
            pltpu.VMEM((m_per, kh), jnp.bfloat16),
            pltpu.VMEM((m_per, kh), jnp.bfloat16),
            pltpu.VMEM((m_per, n), jnp.float32),
            pltpu.SMEM((1,), jnp.float32),
            pltpu.VMEM((8, 128), jnp.float32),
            pltpu.VMEM((N_DEV - 1, 8, 128), jnp.float32),
            pltpu.VMEM((N_DEV - 1, m_per, n_per), jnp.int8),
            pltpu.VMEM((N_DEV - 1, m_per, n_per), jnp.int8),
            pltpu.SemaphoreType.DMA((N_DEV - 1,)),
            pltpu.SemaphoreType.DMA((N_DEV - 1,)),
            pltpu.SemaphoreType.DMA((N_DEV - 1,)),
            pltpu.SemaphoreType.DMA((N_DEV - 1,)),
        ],
        compiler_params=pltpu.CompilerParams(
            dimension_semantics=("arbitrary",),
            collective_id=None if variant == "v3mm" else 0,
            vmem_limit_bytes=vmem_mb * 1024 * 1024,
        ),
    )(x, x, w_mat, w_mat)


def _kernel_v4(x, w_mat, variant):
    m_per, k = x.shape
    _, n = w_mat.shape
    n_per = n // N_DEV
    kq = k // 4

    no_comm = variant in ("v4mm", "v4conv", "v4f32")
    if variant in ("v4f32", "v5", "v6"):
        xb_shape = pltpu.VMEM((1, 8, 128), jnp.bfloat16)
    else:
        xb_shape = pltpu.VMEM((4, m_per, kq), jnp.bfloat16)

    def body(x1_ref, x2_ref, x3_ref, x4_ref,
             w1_ref, w2_ref, w3_ref, w4_ref, out_ref,
             xb_ref, wb_ref, y_ref, amax_sm, amax_tx_ref, amax_rx_ref,
             q_tx_ref, q_rx_ref, amax_send_sems, amax_recv_sems,
             data_send_sems, data_recv_sems):
        s = pl.program_id(0)
        my = lax.axis_index("i")
        barrier = None if no_comm else pltpu.get_barrier_semaphore()
        x_refs = [x1_ref, x2_ref, x3_ref, x4_ref]
        w_refs = [w1_ref, w2_ref, w3_ref, w4_ref]

        @pl.when(s == 0)
        def _prologue():
            if barrier is not None:
                for d in range(1, N_DEV):
                    pl.semaphore_signal(
                        barrier, inc=1,
                        device_id=((my + d) % N_DEV,),
                        device_id_type=pl.DeviceIdType.MESH,
                    )
            if variant not in ("v4f32", "v5", "v6"):
                for i in range(4):
                    xb_ref[i] = x_refs[i][...].astype(jnp.bfloat16)

        if variant == "v4conv":
            for i in range(4):
                wb_ref[...] = w_refs[i][...].astype(jnp.bfloat16)
            partial = wb_ref[...].astype(jnp.float32)
        elif variant in ("v4f32", "v5", "v6"):
            partial = jnp.dot(
                x_refs[0][...], w_refs[0][...],
                preferred_element_type=jnp.float32,
            )
            for i in range(1, 4):
                partial += jnp.dot(
                    x_refs[i][...], w_refs[i][...],
                    preferred_element_type=jnp.float32,
                )
        else:
            partial = jnp.dot(
                xb_ref[0], w_refs[0][...].astype(jnp.bfloat16),
                preferred_element_type=jnp.float32,
            )
            for i in range(1, 4):
                partial += jnp.dot(
                    xb_ref[i], w_refs[i][...].astype(jnp.bfloat16),
                    preferred_element_type=jnp.float32,
                )
        y_ref[:, pl.ds(s * n_per, n_per)] = partial
        pmax = jnp.max(jnp.abs(partial))

        @pl.when(s == 0)
        def _amax_init():
            amax_sm[0] = pmax

        @pl.when(s != 0)
        def _amax_acc():
            amax_sm[0] = jnp.maximum(amax_sm[0], pmax)

        @pl.when(s == N_DEV - 1)
        def _epilogue():
            if no_comm:
                inv0 = 127.0 / amax_sm[0]
                scale0 = amax_sm[0] * (1.0 / 127.0)
                for blk in range(N_DEV):
                    qb = jnp.clip(
                        jnp.round(
                            y_ref[:, pl.ds(blk * n_per, n_per)] * inv0
                        ), -127.0, 127.0,
                    )
                    out_ref[pl.ds(blk * m_per, m_per), :] = (
                        qb * scale0
                    ).astype(jnp.bfloat16)
                return

            pl.semaphore_wait(barrier, N_DEV - 1)

            local_amax = amax_sm[0]
            amax_tx_ref[...] = jnp.full((8, 128), local_amax, jnp.float32)
            amax_rdmas = []
            for d in range(1, N_DEV):
                rdma = pltpu.make_async_remote_copy(
                    src_ref=amax_tx_ref,
                    dst_ref=amax_rx_ref.at[d - 1],
                    send_sem=amax_send_sems.at[d - 1],
                    recv_sem=amax_recv_sems.at[d - 1],
                    device_id=((my + d) % N_DEV,),
                    device_id_type=pl.DeviceIdType.MESH,
                )
                rdma.start()
                amax_rdmas.append(rdma)
            for rdma in amax_rdmas:
                rdma.wait()
            g_amax = local_amax
            for d in range(1, N_DEV):
                g_amax = jnp.maximum(g_amax, amax_rx_ref[d - 1, 0, 0])
            scale = g_amax * (1.0 / 127.0)
            inv_scale = 127.0 / g_amax

            if variant == "v6":
                mh = m_per // 2
                data_rdmas = {}
                for c in range(2):
                    rows = pl.ds(c * mh, mh)
                    for d in (2, 1, 3):
                        tgt = (my + d) % N_DEV
                        blk = y_ref[rows, pl.ds(tgt * n_per, n_per)]
                        q_tx_ref[d - 1, rows, :] = jnp.clip(
                            jnp.round(blk * inv_scale), -127.0, 127.0
                        ).astype(jnp.int8)
                        rdma = pltpu.make_async_remote_copy(
                            src_ref=q_tx_ref.at[d - 1, rows, :],
                            dst_ref=q_rx_ref.at[d - 1, rows, :],
                            send_sem=data_send_sems.at[d - 1, c],
                            recv_sem=data_recv_sems.at[d - 1, c],
                            device_id=(tgt,),
                            device_id_type=pl.DeviceIdType.MESH,
                        )
                        rdma.start()
                        data_rdmas[(d, c)] = rdma

                own = y_ref[:, pl.ds(my * n_per, n_per)]
                q_own = jnp.clip(jnp.round(own * inv_scale), -127.0, 127.0)
                out_ref[pl.ds(my * m_per, m_per), :] = (
                    q_own * scale
                ).astype(jnp.bfloat16)

                for c in range(2):
                    rows = pl.ds(c * mh, mh)
                    for d in range(1, N_DEV):
                        src = (my - d) % N_DEV
                        data_rdmas[(d, c)].wait()
                        out_ref[pl.ds(src * m_per + c * mh, mh), :] = (
                            q_rx_ref[d - 1, rows, :].astype(jnp.float32)
                            * scale
                        ).astype(jnp.bfloat16)
                return

            data_rdmas = {}
            for d in (2, 1, 3):
                tgt = (my + d) % N_DEV
                blk = y_ref[:, pl.ds(tgt * n_per, n_per)]
                q_tx_ref[d - 1] = jnp.clip(
                    jnp.round(blk * inv_scale), -127.0, 127.0
                ).astype(jnp.int8)
                rdma = pltpu.make_async_remote_copy(
                    src_ref=q_tx_ref.at[d - 1],
                    dst_ref=q_rx_ref.at[d - 1],
                    send_sem=data_send_sems.at[d - 1, 0],
                    recv_sem=data_recv_sems.at[d - 1, 0],
                    device_id=(tgt,),
                    device_id_type=pl.DeviceIdType.MESH,
                )
                rdma.start()
                data_rdmas[d] = rdma

            own = y_ref[:, pl.ds(my * n_per, n_per)]
            q_own = jnp.clip(jnp.round(own * inv_scale), -127.0, 127.0)
            out_ref[pl.ds(my * m_per, m_per), :] = (
                q_own * scale
            ).astype(jnp.bfloat16)

            for d in range(1, N_DEV):
                src = (my - d) % N_DEV
                data_rdmas[d].wait()
                out_ref[pl.ds(src * m_per, m_per), :] = (
                    q_rx_ref[d - 1].astype(jnp.float32) * scale
                ).astype(jnp.bfloat16)

    return pl.pallas_call(
        body,
        grid=(N_DEV,),
        out_shape=jax.ShapeDtypeStruct((N_DEV * m_per, n_per), jnp.bfloat16),
        in_specs=(
            [pl.BlockSpec((m_per, kq), (lambda s, i=i: (0, i)))
             for i in range(4)]
            + [pl.BlockSpec((kq, n_per), (lambda s, i=i: (i, s)))
               for i in range(4)]
        ),
        out_specs=pl.BlockSpec((N_DEV * m_per, n_per), lambda s: (0, 0)),
        scratch_shapes=[
            xb_shape,
            pltpu.VMEM((kq, n_per), jnp.bfloat16),
            pltpu.VMEM((m_per, n), jnp.float32),
            pltpu.SMEM((1,), jnp.float32),
            pltpu.VMEM((8, 128), jnp.float32),
            pltpu.VMEM((N_DEV - 1, 8, 128), jnp.float32),
            pltpu.VMEM((N_DEV - 1, m_per, n_per), jnp.int8),
            pltpu.VMEM((N_DEV - 1, m_per, n_per), jnp.int8),
            pltpu.SemaphoreType.DMA((N_DEV - 1,)),
            pltpu.SemaphoreType.DMA((N_DEV - 1,)),
            pltpu.SemaphoreType.DMA((N_DEV - 1, 2)),
            pltpu.SemaphoreType.DMA((N_DEV - 1, 2)),
        ],
        compiler_params=pltpu.CompilerParams(
            dimension_semantics=("arbitrary",),
            collective_id=None if no_comm else 0,
            vmem_limit_bytes=100 * 1024 * 1024,
        ),
    )(x, x, x, x, w_mat, w_mat, w_mat, w_mat)


def _kernel_v7(x, w_mat, variant):
    m_per, k = x.shape
    _, n = w_mat.shape
    n_per = n // N_DEV
    kq = k // 4

    def body(x1_ref, x2_ref, x3_ref, x4_ref,
             w1_ref, w2_ref, w3_ref, w4_ref, out_ref,
             y_ref, txb_ref, rxb_ref, amax_sm, amax_tx_ref, amax_rx_ref,
             tx_sems, rx_sems, amax_send_sems, amax_recv_sems):
        s = pl.program_id(0)
        my = lax.axis_index("i")
        barrier = pltpu.get_barrier_semaphore()
        x_refs = [x1_ref, x2_ref, x3_ref, x4_ref]
        w_refs = [w1_ref, w2_ref, w3_ref, w4_ref]

        @pl.when(s == 0)
        def _entry_barrier():
            for d in range(1, N_DEV):
                pl.semaphore_signal(
                    barrier, inc=1,
                    device_id=((my + d) % N_DEV,),
                    device_id_type=pl.DeviceIdType.MESH,
                )
            pl.semaphore_wait(barrier, N_DEV - 1)

        partial = jnp.dot(
            x_refs[0][...], w_refs[0][...],
            preferred_element_type=jnp.float32,
        )
        for i in range(1, 4):
            partial += jnp.dot(
                x_refs[i][...], w_refs[i][...],
                preferred_element_type=jnp.float32,
            )
        y_ref[:, pl.ds(s * n_per, n_per)] = partial
        pmax = jnp.max(jnp.abs(partial))

        @pl.when(s == 0)
        def _amax_init():
            amax_sm[0] = pmax

        @pl.when(s != 0)
        def _amax_acc():
            amax_sm[0] = jnp.maximum(amax_sm[0], pmax)

        @pl.when(s != my)
        def _send_panel():
            txb_ref[s] = partial.astype(jnp.bfloat16)
            rdma = pltpu.make_async_remote_copy(
                src_ref=txb_ref.at[s],
                dst_ref=rxb_ref.at[my],
                send_sem=tx_sems.at[s],
                recv_sem=rx_sems.at[my],
                device_id=(s,),
                device_id_type=pl.DeviceIdType.MESH,
            )
            rdma.start()

        @pl.when(s == N_DEV - 1)
        def _epilogue():
            local_amax = amax_sm[0]
            amax_tx_ref[...] = jnp.full((8, 128), local_amax, jnp.float32)
            amax_rdmas = []
            for d in range(1, N_DEV):
                rdma = pltpu.make_async_remote_copy(
                    src_ref=amax_tx_ref,
                    dst_ref=amax_rx_ref.at[d - 1],
                    send_sem=amax_send_sems.at[d - 1],
                    recv_sem=amax_recv_sems.at[d - 1],
                    device_id=((my + d) % N_DEV,),
                    device_id_type=pl.DeviceIdType.MESH,
                )
                rdma.start()
                amax_rdmas.append(rdma)
            for rdma in amax_rdmas:
                rdma.wait()
            g_amax = local_amax
            for d in range(1, N_DEV):
                g_amax = jnp.maximum(g_amax, amax_rx_ref[d - 1, 0, 0])
            scale = g_amax * (1.0 / 127.0)
            inv_scale = 127.0 / g_amax

            own = y_ref[:, pl.ds(my * n_per, n_per)]
            q_own = jnp.clip(jnp.round(own * inv_scale), -127.0, 127.0)
            out_ref[pl.ds(my * m_per, m_per), :] = (
                q_own * scale
            ).astype(jnp.bfloat16)

            for d in range(1, N_DEV):
                src = (my - d) % N_DEV
                recv = pltpu.make_async_remote_copy(
                    src_ref=txb_ref.at[0],
                    dst_ref=rxb_ref.at[src],
                    send_sem=tx_sems.at[0],
                    recv_sem=rx_sems.at[src],
                    device_id=(src,),
                    device_id_type=pl.DeviceIdType.MESH,
                )
                recv.wait_recv()
                blk = rxb_ref[src].astype(jnp.float32)
                qb = jnp.clip(jnp.round(blk * inv_scale), -127.0, 127.0)
                out_ref[pl.ds(src * m_per, m_per), :] = (
                    qb * scale
                ).astype(jnp.bfloat16)

            for p in range(N_DEV):
                @pl.when(p != my)
                def _drain(p=p):
                    send = pltpu.make_async_remote_copy(
                        src_ref=txb_ref.at[p],
                        dst_ref=rxb_ref.at[my],
                        send_sem=tx_sems.at[p],
                        recv_sem=rx_sems.at[my],
                        device_id=(p,),
                        device_id_type=pl.DeviceIdType.MESH,
                    )
                    send.wait_send()

    return pl.pallas_call(
        body,
        grid=(N_DEV,),
        out_shape=jax.ShapeDtypeStruct((N_DEV * m_per, n_per), jnp.bfloat16),
        in_specs=(
            [pl.BlockSpec((m_per, kq), (lambda s, i=i: (0, i)))
             for i in range(4)]
            + [pl.BlockSpec((kq, n_per), (lambda s, i=i: (i, s)))
               for i in range(4)]
        ),
        out_specs=pl.BlockSpec((N_DEV * m_per, n_per), lambda s: (0, 0)),
        scratch_shapes=[
            pltpu.VMEM((m_per, n), jnp.float32),
            pltpu.VMEM((N_DEV, m_per, n_per), jnp.bfloat16),
            pltpu.VMEM((N_DEV, m_per, n_per), jnp.bfloat16),
            pltpu.SMEM((1,), jnp.float32),
            pltpu.VMEM((8, 128), jnp.float32),
            pltpu.VMEM((N_DEV - 1, 8, 128), jnp.float32),
            pltpu.SemaphoreType.DMA((N_DEV,)),
            pltpu.SemaphoreType.DMA((N_DEV,)),
            pltpu.SemaphoreType.DMA((N_DEV - 1,)),
            pltpu.SemaphoreType.DMA((N_DEV - 1,)),
        ],
        compiler_params=pltpu.CompilerParams(
            dimension_semantics=("arbitrary",),
            collective_id=0,
            vmem_limit_bytes=100 * 1024 * 1024,
        ),
    )(x, x, x, x, w_mat, w_mat, w_mat, w_mat)


def kernel(x, w_mat, variant="v7", kc=KC):
    if variant == "v7":
        return _kernel_v7(x, w_mat, variant)
    m_per, k = x.shape
    _, n = w_mat.shape
    n_per = n // N_DEV
    nsteps = k // kc

    if variant in ("stream4", "mm4"):
        return _kernel4(x, w_mat, variant, kc)
    if variant.startswith("v3"):
        return _kernel_v3(x, w_mat, variant, vmem_mb=(kc if kc > 8 else 64))
    if variant.startswith("v4") or variant in ("v5", "v6"):
        return _kernel_v4(x, w_mat, variant)

    def body(x_ref, w_ref, out_ref, acc_ref, amax_tx_ref, amax_rx_ref,
             q_tx_ref, q_rx_ref, amax_send_sems, amax_recv_sems,
             data_send_sems, data_recv_sems):
        step = pl.program_id(0)
        my = lax.axis_index("i")

        if variant not in ("gemm", "mm"):
            @pl.when(step == 0)
            def _entry_barrier():
                barrier = pltpu.get_barrier_semaphore()
                for d in range(1, N_DEV):
                    pl.semaphore_signal(
                        barrier, inc=1,
                        device_id=((my + d) % N_DEV,),
                        device_id_type=pl.DeviceIdType.MESH,
                    )
                pl.semaphore_wait(barrier, N_DEV - 1)

        if variant == "stream":
            acc_ref[:kc, :] = w_ref[...]
            acc_ref[:m_per, :kc] += x_ref[...]

            @pl.when(step == nsteps - 1)
            def _stream_out():
                out_ref[...] = jnp.concatenate(
                    [acc_ref[:, :n_per]] * N_DEV, axis=0
                )
            return

        partial = jnp.dot(
            x_ref[...].astype(jnp.bfloat16),
            w_ref[...].astype(jnp.bfloat16),
            preferred_element_type=jnp.float32,
        )

        @pl.when(step == 0)
        def _init_acc():
            acc_ref[...] = partial

        @pl.when(step != 0)
        def _accum():
            acc_ref[...] += partial

        if variant == "mm":
            @pl.when(step == nsteps - 1)
            def _raw_out():
                for blk in range(N_DEV):
                    out_ref[pl.ds(blk * m_per, m_per), :] = (
                        acc_ref[:, pl.ds(blk * n_per, n_per)]
                    )
            return

        @pl.when(step == nsteps - 1)
        def _epilogue():
            local_amax = jnp.max(jnp.abs(acc_ref[...]))

            if variant == "gemm":
                g_amax = local_amax
            else:
                amax_tx_ref[...] = jnp.full((8, 128), local_amax, jnp.float32)
                amax_rdmas = []
                for d in range(1, N_DEV):
                    rdma = pltpu.make_async_remote_copy(
                        src_ref=amax_tx_ref,
                        dst_ref=amax_rx_ref.at[d - 1],
                        send_sem=amax_send_sems.at[d - 1],
                        recv_sem=amax_recv_sems.at[d - 1],
                        device_id=((my + d) % N_DEV,),
                        device_id_type=pl.DeviceIdType.MESH,
                    )
                    rdma.start()
                    amax_rdmas.append(rdma)
                for rdma in amax_rdmas:
                    rdma.wait()
                g_amax = local_amax
                for d in range(1, N_DEV):
                    g_amax = jnp.maximum(g_amax, amax_rx_ref[d - 1, 0, 0])

            scale = g_amax * (1.0 / 127.0)
            inv_scale = 127.0 / g_amax

            data_rdmas = []
            for d in range(1, N_DEV):
                tgt = (my + d) % N_DEV
                blk = acc_ref[:, pl.ds(tgt * n_per, n_per)]
                q_tx_ref[d - 1] = jnp.clip(
                    jnp.round(blk * inv_scale), -127.0, 127.0
                ).astype(jnp.int8)
                if variant == "full":
                    rdma = pltpu.make_async_remote_copy(
                        src_ref=q_tx_ref.at[d - 1],
                        dst_ref=q_rx_ref.at[d - 1],
                        send_sem=data_send_sems.at[d - 1],
                        recv_sem=data_recv_sems.at[d - 1],
                        device_id=(tgt,),
                        device_id_type=pl.DeviceIdType.MESH,
                    )
                    rdma.start()
                    data_rdmas.append(rdma)

            own = acc_ref[:, pl.ds(my * n_per, n_per)]
            q_own = jnp.clip(jnp.round(own * inv_scale), -127.0, 127.0)
            out_ref[pl.ds(my * m_per, m_per), :] = q_own * scale

            for d in range(1, N_DEV):
                src = (my - d) % N_DEV
                if variant == "full":
                    data_rdmas[d - 1].wait()
                    q_src = q_rx_ref[d - 1]
                else:
                    q_src = q_tx_ref[d - 1]
                out_ref[pl.ds(src * m_per, m_per), :] = (
                    q_src.astype(jnp.float32) * scale
                )

    return pl.pallas_call(
        body,
        grid=(nsteps,),
        out_shape=jax.ShapeDtypeStruct((N_DEV * m_per, n_per), jnp.float32),
        in_specs=[
            pl.BlockSpec((m_per, kc), lambda s: (0, s)),
            pl.BlockSpec((kc, n), lambda s: (s, 0)),
        ],
        out_specs=pl.BlockSpec((N_DEV * m_per, n_per), lambda s: (0, 0)),
        scratch_shapes=[
            pltpu.VMEM((m_per, n), jnp.float32),
            pltpu.VMEM((8, 128), jnp.float32),
            pltpu.VMEM((N_DEV - 1, 8, 128), jnp.float32),
            pltpu.VMEM((N_DEV - 1, m_per, n_per), jnp.int8),
            pltpu.VMEM((N_DEV - 1, m_per, n_per), jnp.int8),
            pltpu.SemaphoreType.DMA((N_DEV - 1,)),
            pltpu.SemaphoreType.DMA((N_DEV - 1,)),
            pltpu.SemaphoreType.DMA((N_DEV - 1,)),
            pltpu.SemaphoreType.DMA((N_DEV - 1,)),
        ],
        compiler_params=pltpu.CompilerParams(
            dimension_semantics=("arbitrary",),
            collective_id=None if variant in ("gemm", "mm") else 0,
            vmem_limit_bytes=64 * 1024 * 1024,
        ),
    )(x, w_mat)


# device time: 55372 ns/iter; 1.1500x vs baseline; 1.1500x over previous
import jax
import jax.numpy as jnp
from jax import lax
from jax.experimental import pallas as pl
from jax.experimental.pallas import tpu as pltpu

N_DEV = 4
KC = 512


def _kernel4(x, w_mat, variant, kc):
    m_per, k = x.shape
    _, n = w_mat.shape
    n_per = n // N_DEV
    kh = k // 2
    nsteps = kh // kc

    def body(x1_ref, x2_ref, w1_ref, w2_ref, out_ref, acc_ref):
        step = pl.program_id(0)

        if variant == "stream4":
            acc_ref[:kc, :] = w1_ref[...]
            acc_ref[kc:2 * kc, :] = w2_ref[...]
            acc_ref[:m_per, :kc] += x1_ref[...]
            acc_ref[:m_per, kc:2 * kc] += x2_ref[...]

            @pl.when(step == nsteps - 1)
            def _stream_out():
                out_ref[...] = jnp.concatenate(
                    [acc_ref[:m_per, :n_per]] * N_DEV, axis=0
                )
            return

        partial = jnp.dot(
            x1_ref[...].astype(jnp.bfloat16),
            w1_ref[...].astype(jnp.bfloat16),
            preferred_element_type=jnp.float32,
        ) + jnp.dot(
            x2_ref[...].astype(jnp.bfloat16),
            w2_ref[...].astype(jnp.bfloat16),
            preferred_element_type=jnp.float32,
        )

        @pl.when(step == 0)
        def _init_acc():
            acc_ref[...] = partial

        @pl.when(step != 0)
        def _accum():
            acc_ref[...] += partial

        @pl.when(step == nsteps - 1)
        def _raw_out():
            for blk in range(N_DEV):
                out_ref[pl.ds(blk * m_per, m_per), :] = (
                    acc_ref[:, pl.ds(blk * n_per, n_per)]
                )

    return pl.pallas_call(
        body,
        grid=(nsteps,),
        out_shape=jax.ShapeDtypeStruct((N_DEV * m_per, n_per), jnp.float32),
        in_specs=[
            pl.BlockSpec((m_per, kc), lambda s: (0, s)),
            pl.BlockSpec((m_per, kc), lambda s: (0, s + nsteps)),
            pl.BlockSpec((kc, n), lambda s: (s, 0)),
            pl.BlockSpec((kc, n), lambda s: (s + nsteps, 0)),
        ],
        out_specs=pl.BlockSpec((N_DEV * m_per, n_per), lambda s: (0, 0)),
        scratch_shapes=[
            pltpu.VMEM((m_per, n), jnp.float32),
        ],
        compiler_params=pltpu.CompilerParams(
            dimension_semantics=("arbitrary",),
            vmem_limit_bytes=64 * 1024 * 1024,
        ),
    )(x, x, w_mat, w_mat)


def _kernel_v3(x, w_mat, variant, vmem_mb=64):
    m_per, k = x.shape
    _, n = w_mat.shape
    n_per = n // N_DEV
    kh = k // 2

    def body(x1_ref, x2_ref, w1_ref, w2_ref, out_ref,
             xb1_ref, xb2_ref, y_ref, amax_sm, amax_tx_ref, amax_rx_ref,
             q_tx_ref, q_rx_ref, amax_send_sems, amax_recv_sems,
             data_send_sems, data_recv_sems):
        s = pl.program_id(0)
        my = lax.axis_index("i")

        @pl.when(s == 0)
        def _prologue():
            if variant != "v3mm":
                barrier = pltpu.get_barrier_semaphore()
                for d in range(1, N_DEV):
                    pl.semaphore_signal(
                        barrier, inc=1,
                        device_id=((my + d) % N_DEV,),
                        device_id_type=pl.DeviceIdType.MESH,
                    )
                pl.semaphore_wait(barrier, N_DEV - 1)
            xb1_ref[...] = x1_ref[...].astype(jnp.bfloat16)
            xb2_ref[...] = x2_ref[...].astype(jnp.bfloat16)

        partial = jnp.dot(
            xb1_ref[...], w1_ref[...].astype(jnp.bfloat16),
            preferred_element_type=jnp.float32,
        ) + jnp.dot(
            xb2_ref[...], w2_ref[...].astype(jnp.bfloat16),
            preferred_element_type=jnp.float32,
        )
        y_ref[:, pl.ds(s * n_per, n_per)] = partial
        pmax = jnp.max(jnp.abs(partial))

        @pl.when(s == 0)
        def _amax_init():
            amax_sm[0] = pmax

        @pl.when(s != 0)
        def _amax_acc():
            amax_sm[0] = jnp.maximum(amax_sm[0], pmax)

        @pl.when(s == N_DEV - 1)
        def _epilogue():
            local_amax = amax_sm[0]
            if variant in ("v3mm", "v3sync"):
                scale0 = local_amax * (1.0 / 127.0)
                inv0 = 127.0 / local_amax
                for blk in range(N_DEV):
                    q = jnp.clip(
                        jnp.round(
                            y_ref[:, pl.ds(blk * n_per, n_per)] * inv0
                        ), -127.0, 127.0,
                    )
                    out_ref[pl.ds(blk * m_per, m_per), :] = (
                        q * scale0
                    ).astype(jnp.bfloat16)
                return
            amax_tx_ref[...] = jnp.full((8, 128), local_amax, jnp.float32)
            amax_rdmas = []
            for d in range(1, N_DEV):
                rdma = pltpu.make_async_remote_copy(
                    src_ref=amax_tx_ref,
                    dst_ref=amax_rx_ref.at[d - 1],
                    send_sem=amax_send_sems.at[d - 1],
                    recv_sem=amax_recv_sems.at[d - 1],
                    device_id=((my + d) % N_DEV,),
                    device_id_type=pl.DeviceIdType.MESH,
                )
                rdma.start()
                amax_rdmas.append(rdma)
            for rdma in amax_rdmas:
                rdma.wait()
            g_amax = local_amax
            for d in range(1, N_DEV):
                g_amax = jnp.maximum(g_amax, amax_rx_ref[d - 1, 0, 0])
            scale = g_amax * (1.0 / 127.0)
            inv_scale = 127.0 / g_amax

            data_rdmas = []
            for d in range(1, N_DEV):
                tgt = (my + d) % N_DEV
                blk = y_ref[:, pl.ds(tgt * n_per, n_per)]
                q_tx_ref[d - 1] = jnp.clip(
                    jnp.round(blk * inv_scale), -127.0, 127.0
                ).astype(jnp.int8)
                if variant != "v3noa2a":
                    rdma = pltpu.make_async_remote_copy(
                        src_ref=q_tx_ref.at[d - 1],
                        dst_ref=q_rx_ref.at[d - 1],
                        send_sem=data_send_sems.at[d - 1],
                        recv_sem=data_recv_sems.at[d - 1],
                        device_id=(tgt,),
                        device_id_type=pl.DeviceIdType.MESH,
                    )
                    rdma.start()
                    data_rdmas.append(rdma)

            own = y_ref[:, pl.ds(my * n_per, n_per)]
            q_own = jnp.clip(jnp.round(own * inv_scale), -127.0, 127.0)
            out_ref[pl.ds(my * m_per, m_per), :] = (
                q_own * scale
            ).astype(jnp.bfloat16)

            for d in range(1, N_DEV):
                src = (my - d) % N_DEV
                if variant != "v3noa2a":
                    data_rdmas[d - 1].wait()
                    q_src = q_rx_ref[d - 1]
                else:
                    q_src = q_tx_ref[d - 1]
                out_ref[pl.ds(src * m_per, m_per), :] = (
                    q_src.astype(jnp.float32) * scale
                ).astype(jnp.bfloat16)

    return pl.pallas_call(
        body,
        grid=(N_DEV,),
        out_shape=jax.ShapeDtypeStruct((N_DEV * m_per, n_per), jnp.bfloat16),
        in_specs=[
            pl.BlockSpec((m_per, kh), lambda s: (0, 0)),
            pl.BlockSpec((m_per, kh), lambda s: (0, 1)),
            pl.BlockSpec((kh, n_per), lambda s: (0, s)),
            pl.BlockSpec((kh, n_per), lambda s: (1, s)),
        ],
        out_specs=pl.BlockSpec((N_DEV * m_per, n_per), lambda s: (0, 0)),
        scratch_shapes=[
            pltpu.VMEM((m_per, kh), jnp.bfloat16),
            pltpu.VMEM((m_per, kh), jnp.bfloat16),
            pltpu.VMEM((m_per, n), jnp.float32),
            pltpu.SMEM((1,), jnp.float32),
            pltpu.VMEM((8, 128), jnp.float32),
            pltpu.VMEM((N_DEV - 1, 8, 128), jnp.float32),
            pltpu.VMEM((N_DEV - 1, m_per, n_per), jnp.int8),
            pltpu.VMEM((N_DEV - 1, m_per, n_per), jnp.int8),
            pltpu.SemaphoreType.DMA((N_DEV - 1,)),
            pltpu.SemaphoreType.DMA((N_DEV - 1,)),
            pltpu.SemaphoreType.DMA((N_DEV - 1,)),
            pltpu.SemaphoreType.DMA((N_DEV - 1,)),
        ],
        compiler_params=pltpu.CompilerParams(
            dimension_semantics=("arbitrary",),
            collective_id=None if variant == "v3mm" else 0,
            vmem_limit_bytes=vmem_mb * 1024 * 1024,
        ),
    )(x, x, w_mat, w_mat)


def _kernel_v4(x, w_mat, variant):
    m_per, k = x.shape
    _, n = w_mat.shape
    n_per = n // N_DEV
    kq = k // 4

    no_comm = variant in ("v4mm", "v4conv", "v4f32")
    if variant in ("v4f32", "v5", "v6"):
        xb_shape = pltpu.VMEM((1, 8, 128), jnp.bfloat16)
    else:
        xb_shape = pltpu.VMEM((4, m_per, kq), jnp.bfloat16)

    def body(x1_ref, x2_ref, x3_ref, x4_ref,
             w1_ref, w2_ref, w3_ref, w4_ref, out_ref,
             xb_ref, wb_ref, y_ref, amax_sm, amax_tx_ref, amax_rx_ref,
             q_tx_ref, q_rx_ref, amax_send_sems, amax_recv_sems,
             data_send_sems, data_recv_sems):
        s = pl.program_id(0)
        my = lax.axis_index("i")
        barrier = None if no_comm else pltpu.get_barrier_semaphore()
        x_refs = [x1_ref, x2_ref, x3_ref, x4_ref]
        w_refs = [w1_ref, w2_ref, w3_ref, w4_ref]

        @pl.when(s == 0)
        def _prologue():
            if barrier is not None:
                for d in range(1, N_DEV):
                    pl.semaphore_signal(
                        barrier, inc=1,
                        device_id=((my + d) % N_DEV,),
                        device_id_type=pl.DeviceIdType.MESH,
                    )
            if variant not in ("v4f32", "v5", "v6"):
                for i in range(4):
                    xb_ref[i] = x_refs[i][...].astype(jnp.bfloat16)

        if variant == "v4conv":
            for i in range(4):
                wb_ref[...] = w_refs[i][...].astype(jnp.bfloat16)
            partial = wb_ref[...].astype(jnp.float32)
        elif variant in ("v4f32", "v5", "v6"):
            partial = jnp.dot(
                x_refs[0][...], w_refs[0][...],
                preferred_element_type=jnp.float32,
            )
            for i in range(1, 4):
                partial += jnp.dot(
                    x_refs[i][...], w_refs[i][...],
                    preferred_element_type=jnp.float32,
                )
        else:
            partial = jnp.dot(
                xb_ref[0], w_refs[0][...].astype(jnp.bfloat16),
                preferred_element_type=jnp.float32,
            )
            for i in range(1, 4):
                partial += jnp.dot(
                    xb_ref[i], w_refs[i][...].astype(jnp.bfloat16),
                    preferred_element_type=jnp.float32,
                )
        y_ref[:, pl.ds(s * n_per, n_per)] = partial
        pmax = jnp.max(jnp.abs(partial))

        @pl.when(s == 0)
        def _amax_init():
            amax_sm[0] = pmax

        @pl.when(s != 0)
        def _amax_acc():
            amax_sm[0] = jnp.maximum(amax_sm[0], pmax)

        @pl.when(s == N_DEV - 1)
        def _epilogue():
            if no_comm:
                inv0 = 127.0 / amax_sm[0]
                scale0 = amax_sm[0] * (1.0 / 127.0)
                for blk in range(N_DEV):
                    qb = jnp.clip(
                        jnp.round(
                            y_ref[:, pl.ds(blk * n_per, n_per)] * inv0
                        ), -127.0, 127.0,
                    )
                    out_ref[pl.ds(blk * m_per, m_per), :] = (
                        qb * scale0
                    ).astype(jnp.bfloat16)
                return

            pl.semaphore_wait(barrier, N_DEV - 1)

            local_amax = amax_sm[0]
            amax_tx_ref[...] = jnp.full((8, 128), local_amax, jnp.float32)
            amax_rdmas = []
            for d in range(1, N_DEV):
                rdma = pltpu.make_async_remote_copy(
                    src_ref=amax_tx_ref,
                    dst_ref=amax_rx_ref.at[d - 1],
                    send_sem=amax_send_sems.at[d - 1],
                    recv_sem=amax_recv_sems.at[d - 1],
                    device_id=((my + d) % N_DEV,),
                    device_id_type=pl.DeviceIdType.MESH,
                )
                rdma.start()
                amax_rdmas.append(rdma)
            for rdma in amax_rdmas:
                rdma.wait()
            g_amax = local_amax
            for d in range(1, N_DEV):
                g_amax = jnp.maximum(g_amax, amax_rx_ref[d - 1, 0, 0])
            scale = g_amax * (1.0 / 127.0)
            inv_scale = 127.0 / g_amax

            if variant == "v6":
                mh = m_per // 2
                data_rdmas = {}
                for c in range(2):
                    rows = pl.ds(c * mh, mh)
                    for d in (2, 1, 3):
                        tgt = (my + d) % N_DEV
                        blk = y_ref[rows, pl.ds(tgt * n_per, n_per)]
                        q_tx_ref[d - 1, rows, :] = jnp.clip(
                            jnp.round(blk * inv_scale), -127.0, 127.0
                        ).astype(jnp.int8)
                        rdma = pltpu.make_async_remote_copy(
                            src_ref=q_tx_ref.at[d - 1, rows, :],
                            dst_ref=q_rx_ref.at[d - 1, rows, :],
                            send_sem=data_send_sems.at[d - 1, c],
                            recv_sem=data_recv_sems.at[d - 1, c],
                            device_id=(tgt,),
                            device_id_type=pl.DeviceIdType.MESH,
                        )
                        rdma.start()
                        data_rdmas[(d, c)] = rdma

                own = y_ref[:, pl.ds(my * n_per, n_per)]
                q_own = jnp.clip(jnp.round(own * inv_scale), -127.0, 127.0)
                out_ref[pl.ds(my * m_per, m_per), :] = (
                    q_own * scale
                ).astype(jnp.bfloat16)

                for c in range(2):
                    rows = pl.ds(c * mh, mh)
                    for d in range(1, N_DEV):
                        src = (my - d) % N_DEV
                        data_rdmas[(d, c)].wait()
                        out_ref[pl.ds(src * m_per + c * mh, mh), :] = (
                            q_rx_ref[d - 1, rows, :].astype(jnp.float32)
                            * scale
                        ).astype(jnp.bfloat16)
                return

            data_rdmas = {}
            for d in (2, 1, 3):
                tgt = (my + d) % N_DEV
                blk = y_ref[:, pl.ds(tgt * n_per, n_per)]
                q_tx_ref[d - 1] = jnp.clip(
                    jnp.round(blk * inv_scale), -127.0, 127.0
                ).astype(jnp.int8)
                rdma = pltpu.make_async_remote_copy(
                    src_ref=q_tx_ref.at[d - 1],
                    dst_ref=q_rx_ref.at[d - 1],
                    send_sem=data_send_sems.at[d - 1, 0],
                    recv_sem=data_recv_sems.at[d - 1, 0],
                    device_id=(tgt,),
                    device_id_type=pl.DeviceIdType.MESH,
                )
                rdma.start()
                data_rdmas[d] = rdma

            own = y_ref[:, pl.ds(my * n_per, n_per)]
            q_own = jnp.clip(jnp.round(own * inv_scale), -127.0, 127.0)
            out_ref[pl.ds(my * m_per, m_per), :] = (
                q_own * scale
            ).astype(jnp.bfloat16)

            for d in range(1, N_DEV):
                src = (my - d) % N_DEV
                data_rdmas[d].wait()
                out_ref[pl.ds(src * m_per, m_per), :] = (
                    q_rx_ref[d - 1].astype(jnp.float32) * scale
                ).astype(jnp.bfloat16)

    return pl.pallas_call(
        body,
        grid=(N_DEV,),
        out_shape=jax.ShapeDtypeStruct((N_DEV * m_per, n_per), jnp.bfloat16),
        in_specs=(
            [pl.BlockSpec((m_per, kq), (lambda s, i=i: (0, i)))
             for i in range(4)]
            + [pl.BlockSpec((kq, n_per), (lambda s, i=i: (i, s)))
               for i in range(4)]
        ),
        out_specs=pl.BlockSpec((N_DEV * m_per, n_per), lambda s: (0, 0)),
        scratch_shapes=[
            xb_shape,
            pltpu.VMEM((kq, n_per), jnp.bfloat16),
            pltpu.VMEM((m_per, n), jnp.float32),
            pltpu.SMEM((1,), jnp.float32),
            pltpu.VMEM((8, 128), jnp.float32),
            pltpu.VMEM((N_DEV - 1, 8, 128), jnp.float32),
            pltpu.VMEM((N_DEV - 1, m_per, n_per), jnp.int8),
            pltpu.VMEM((N_DEV - 1, m_per, n_per), jnp.int8),
            pltpu.SemaphoreType.DMA((N_DEV - 1,)),
            pltpu.SemaphoreType.DMA((N_DEV - 1,)),
            pltpu.SemaphoreType.DMA((N_DEV - 1, 2)),
            pltpu.SemaphoreType.DMA((N_DEV - 1, 2)),
        ],
        compiler_params=pltpu.CompilerParams(
            dimension_semantics=("arbitrary",),
            collective_id=None if no_comm else 0,
            vmem_limit_bytes=100 * 1024 * 1024,
        ),
    )(x, x, x, x, w_mat, w_mat, w_mat, w_mat)


def _kernel_v7(x, w_mat, variant):
    m_per, k = x.shape
    _, n = w_mat.shape
    n_per = n // N_DEV
    kq = k // 4

    def w_panel(s):
        if variant == "v8":
            return (lax.axis_index("i") + 1 + s) % N_DEV
        return s

    def body(x1_ref, x2_ref, x3_ref, x4_ref,
             w1_ref, w2_ref, w3_ref, w4_ref, out_ref,
             y_ref, txb_ref, rxb_ref, amax_sm, amax_tx_ref, amax_rx_ref,
             tx_sems, rx_sems, amax_send_sems, amax_recv_sems):
        s = pl.program_id(0)
        my = lax.axis_index("i")
        barrier = pltpu.get_barrier_semaphore()
        x_refs = [x1_ref, x2_ref, x3_ref, x4_ref]
        w_refs = [w1_ref, w2_ref, w3_ref, w4_ref]

        @pl.when(s == 0)
        def _entry_barrier():
            for d in range(1, N_DEV):
                pl.semaphore_signal(
                    barrier, inc=1,
                    device_id=((my + d) % N_DEV,),
                    device_id_type=pl.DeviceIdType.MESH,
                )
            pl.semaphore_wait(barrier, N_DEV - 1)

        p = (my + 1 + s) % N_DEV if variant == "v8" else s
        partial = jnp.dot(
            x_refs[0][...], w_refs[0][...],
            preferred_element_type=jnp.float32,
        )
        for i in range(1, 4):
            partial += jnp.dot(
                x_refs[i][...], w_refs[i][...],
                preferred_element_type=jnp.float32,
            )
        y_ref[:, pl.ds(p * n_per, n_per)] = partial
        pmax = jnp.max(jnp.abs(partial))

        @pl.when(s == 0)
        def _amax_init():
            amax_sm[0] = pmax

        @pl.when(s != 0)
        def _amax_acc():
            amax_sm[0] = jnp.maximum(amax_sm[0], pmax)

        @pl.when(p != my)
        def _send_panel():
            txb_ref[p] = partial.astype(jnp.bfloat16)
            rdma = pltpu.make_async_remote_copy(
                src_ref=txb_ref.at[p],
                dst_ref=rxb_ref.at[my],
                send_sem=tx_sems.at[p],
                recv_sem=rx_sems.at[my],
                device_id=(p,),
                device_id_type=pl.DeviceIdType.MESH,
            )
            rdma.start()

        @pl.when(s == N_DEV - 1)
        def _epilogue():
            local_amax = amax_sm[0]
            amax_tx_ref[...] = jnp.full((8, 128), local_amax, jnp.float32)
            amax_rdmas = []
            for d in range(1, N_DEV):
                rdma = pltpu.make_async_remote_copy(
                    src_ref=amax_tx_ref,
                    dst_ref=amax_rx_ref.at[d - 1],
                    send_sem=amax_send_sems.at[d - 1],
                    recv_sem=amax_recv_sems.at[d - 1],
                    device_id=((my + d) % N_DEV,),
                    device_id_type=pl.DeviceIdType.MESH,
                )
                rdma.start()
                amax_rdmas.append(rdma)
            for rdma in amax_rdmas:
                rdma.wait()
            g_amax = local_amax
            for d in range(1, N_DEV):
                g_amax = jnp.maximum(g_amax, amax_rx_ref[d - 1, 0, 0])
            scale = g_amax * (1.0 / 127.0)
            inv_scale = 127.0 / g_amax

            own = y_ref[:, pl.ds(my * n_per, n_per)]
            q_own = jnp.clip(jnp.round(own * inv_scale), -127.0, 127.0)
            out_ref[pl.ds(my * m_per, m_per), :] = (
                q_own * scale
            ).astype(jnp.bfloat16)

            for d in range(1, N_DEV):
                src = (my - d) % N_DEV
                recv = pltpu.make_async_remote_copy(
                    src_ref=txb_ref.at[0],
                    dst_ref=rxb_ref.at[src],
                    send_sem=tx_sems.at[0],
                    recv_sem=rx_sems.at[src],
                    device_id=(src,),
                    device_id_type=pl.DeviceIdType.MESH,
                )
                recv.wait_recv()
                blk = rxb_ref[src].astype(jnp.float32)
                qb = jnp.clip(jnp.round(blk * inv_scale), -127.0, 127.0)
                out_ref[pl.ds(src * m_per, m_per), :] = (
                    qb * scale
                ).astype(jnp.bfloat16)

            for p in range(N_DEV):
                @pl.when(p != my)
                def _drain(p=p):
                    send = pltpu.make_async_remote_copy(
                        src_ref=txb_ref.at[p],
                        dst_ref=rxb_ref.at[my],
                        send_sem=tx_sems.at[p],
                        recv_sem=rx_sems.at[my],
                        device_id=(p,),
                        device_id_type=pl.DeviceIdType.MESH,
                    )
                    send.wait_send()

    return pl.pallas_call(
        body,
        grid=(N_DEV,),
        out_shape=jax.ShapeDtypeStruct((N_DEV * m_per, n_per), jnp.bfloat16),
        in_specs=(
            [pl.BlockSpec((m_per, kq), (lambda s, i=i: (0, i)))
             for i in range(4)]
            + [pl.BlockSpec((kq, n_per), (lambda s, i=i: (i, w_panel(s))))
               for i in range(4)]
        ),
        out_specs=pl.BlockSpec((N_DEV * m_per, n_per), lambda s: (0, 0)),
        scratch_shapes=[
            pltpu.VMEM((m_per, n), jnp.float32),
            pltpu.VMEM((N_DEV, m_per, n_per), jnp.bfloat16),
            pltpu.VMEM((N_DEV, m_per, n_per), jnp.bfloat16),
            pltpu.SMEM((1,), jnp.float32),
            pltpu.VMEM((8, 128), jnp.float32),
            pltpu.VMEM((N_DEV - 1, 8, 128), jnp.float32),
            pltpu.SemaphoreType.DMA((N_DEV,)),
            pltpu.SemaphoreType.DMA((N_DEV,)),
            pltpu.SemaphoreType.DMA((N_DEV - 1,)),
            pltpu.SemaphoreType.DMA((N_DEV - 1,)),
        ],
        compiler_params=pltpu.CompilerParams(
            dimension_semantics=("arbitrary",),
            collective_id=0,
            vmem_limit_bytes=100 * 1024 * 1024,
        ),
    )(x, x, x, x, w_mat, w_mat, w_mat, w_mat)


def kernel(x, w_mat, variant="v8", kc=KC):
    if variant in ("v7", "v8"):
        return _kernel_v7(x, w_mat, variant)
    m_per, k = x.shape
    _, n = w_mat.shape
    n_per = n // N_DEV
    nsteps = k // kc

    if variant in ("stream4", "mm4"):
        return _kernel4(x, w_mat, variant, kc)
    if variant.startswith("v3"):
        return _kernel_v3(x, w_mat, variant, vmem_mb=(kc if kc > 8 else 64))
    if variant.startswith("v4") or variant in ("v5", "v6"):
        return _kernel_v4(x, w_mat, variant)

    def body(x_ref, w_ref, out_ref, acc_ref, amax_tx_ref, amax_rx_ref,
             q_tx_ref, q_rx_ref, amax_send_sems, amax_recv_sems,
             data_send_sems, data_recv_sems):
        step = pl.program_id(0)
        my = lax.axis_index("i")

        if variant not in ("gemm", "mm"):
            @pl.when(step == 0)
            def _entry_barrier():
                barrier = pltpu.get_barrier_semaphore()
                for d in range(1, N_DEV):
                    pl.semaphore_signal(
                        barrier, inc=1,
                        device_id=((my + d) % N_DEV,),
                        device_id_type=pl.DeviceIdType.MESH,
                    )
                pl.semaphore_wait(barrier, N_DEV - 1)

        if variant == "stream":
            acc_ref[:kc, :] = w_ref[...]
            acc_ref[:m_per, :kc] += x_ref[...]

            @pl.when(step == nsteps - 1)
            def _stream_out():
                out_ref[...] = jnp.concatenate(
                    [acc_ref[:, :n_per]] * N_DEV, axis=0
                )
            return

        partial = jnp.dot(
            x_ref[...].astype(jnp.bfloat16),
            w_ref[...].astype(jnp.bfloat16),
            preferred_element_type=jnp.float32,
        )

        @pl.when(step == 0)
        def _init_acc():
            acc_ref[...] = partial

        @pl.when(step != 0)
        def _accum():
            acc_ref[...] += partial

        if variant == "mm":
            @pl.when(step == nsteps - 1)
            def _raw_out():
                for blk in range(N_DEV):
                    out_ref[pl.ds(blk * m_per, m_per), :] = (
                        acc_ref[:, pl.ds(blk * n_per, n_per)]
                    )
            return

        @pl.when(step == nsteps - 1)
        def _epilogue():
            local_amax = jnp.max(jnp.abs(acc_ref[...]))

            if variant == "gemm":
                g_amax = local_amax
            else:
                amax_tx_ref[...] = jnp.full((8, 128), local_amax, jnp.float32)
                amax_rdmas = []
                for d in range(1, N_DEV):
                    rdma = pltpu.make_async_remote_copy(
                        src_ref=amax_tx_ref,
                        dst_ref=amax_rx_ref.at[d - 1],
                        send_sem=amax_send_sems.at[d - 1],
                        recv_sem=amax_recv_sems.at[d - 1],
                        device_id=((my + d) % N_DEV,),
                        device_id_type=pl.DeviceIdType.MESH,
                    )
                    rdma.start()
                    amax_rdmas.append(rdma)
                for rdma in amax_rdmas:
                    rdma.wait()
                g_amax = local_amax
                for d in range(1, N_DEV):
                    g_amax = jnp.maximum(g_amax, amax_rx_ref[d - 1, 0, 0])

            scale = g_amax * (1.0 / 127.0)
            inv_scale = 127.0 / g_amax

            data_rdmas = []
            for d in range(1, N_DEV):
                tgt = (my + d) % N_DEV
                blk = acc_ref[:, pl.ds(tgt * n_per, n_per)]
                q_tx_ref[d - 1] = jnp.clip(
                    jnp.round(blk * inv_scale), -127.0, 127.0
                ).astype(jnp.int8)
                if variant == "full":
                    rdma = pltpu.make_async_remote_copy(
                        src_ref=q_tx_ref.at[d - 1],
                        dst_ref=q_rx_ref.at[d - 1],
                        send_sem=data_send_sems.at[d - 1],
                        recv_sem=data_recv_sems.at[d - 1],
                        device_id=(tgt,),
                        device_id_type=pl.DeviceIdType.MESH,
                    )
                    rdma.start()
                    data_rdmas.append(rdma)

            own = acc_ref[:, pl.ds(my * n_per, n_per)]
            q_own = jnp.clip(jnp.round(own * inv_scale), -127.0, 127.0)
            out_ref[pl.ds(my * m_per, m_per), :] = q_own * scale

            for d in range(1, N_DEV):
                src = (my - d) % N_DEV
                if variant == "full":
                    data_rdmas[d - 1].wait()
                    q_src = q_rx_ref[d - 1]
                else:
                    q_src = q_tx_ref[d - 1]
                out_ref[pl.ds(src * m_per, m_per), :] = (
                    q_src.astype(jnp.float32) * scale
                )

    return pl.pallas_call(
        body,
        grid=(nsteps,),
        out_shape=jax.ShapeDtypeStruct((N_DEV * m_per, n_per), jnp.float32),
        in_specs=[
            pl.BlockSpec((m_per, kc), lambda s: (0, s)),
            pl.BlockSpec((kc, n), lambda s: (s, 0)),
        ],
        out_specs=pl.BlockSpec((N_DEV * m_per, n_per), lambda s: (0, 0)),
        scratch_shapes=[
            pltpu.VMEM((m_per, n), jnp.float32),
            pltpu.VMEM((8, 128), jnp.float32),
            pltpu.VMEM((N_DEV - 1, 8, 128), jnp.float32),
            pltpu.VMEM((N_DEV - 1, m_per, n_per), jnp.int8),
            pltpu.VMEM((N_DEV - 1, m_per, n_per), jnp.int8),
            pltpu.SemaphoreType.DMA((N_DEV - 1,)),
            pltpu.SemaphoreType.DMA((N_DEV - 1,)),
            pltpu.SemaphoreType.DMA((N_DEV - 1,)),
            pltpu.SemaphoreType.DMA((N_DEV - 1,)),
        ],
        compiler_params=pltpu.CompilerParams(
            dimension_semantics=("arbitrary",),
            collective_id=None if variant in ("gemm", "mm") else 0,
            vmem_limit_bytes=64 * 1024 * 1024,
        ),
    )(x, w_mat)


# device time: 55017 ns/iter; 1.1574x vs baseline; 1.0065x over previous
import jax
import jax.numpy as jnp
from jax import lax
from jax.experimental import pallas as pl
from jax.experimental.pallas import tpu as pltpu

N_DEV = 4
KC = 512


def _kernel4(x, w_mat, variant, kc):
    m_per, k = x.shape
    _, n = w_mat.shape
    n_per = n // N_DEV
    kh = k // 2
    nsteps = kh // kc

    def body(x1_ref, x2_ref, w1_ref, w2_ref, out_ref, acc_ref):
        step = pl.program_id(0)

        if variant == "stream4":
            acc_ref[:kc, :] = w1_ref[...]
            acc_ref[kc:2 * kc, :] = w2_ref[...]
            acc_ref[:m_per, :kc] += x1_ref[...]
            acc_ref[:m_per, kc:2 * kc] += x2_ref[...]

            @pl.when(step == nsteps - 1)
            def _stream_out():
                out_ref[...] = jnp.concatenate(
                    [acc_ref[:m_per, :n_per]] * N_DEV, axis=0
                )
            return

        partial = jnp.dot(
            x1_ref[...].astype(jnp.bfloat16),
            w1_ref[...].astype(jnp.bfloat16),
            preferred_element_type=jnp.float32,
        ) + jnp.dot(
            x2_ref[...].astype(jnp.bfloat16),
            w2_ref[...].astype(jnp.bfloat16),
            preferred_element_type=jnp.float32,
        )

        @pl.when(step == 0)
        def _init_acc():
            acc_ref[...] = partial

        @pl.when(step != 0)
        def _accum():
            acc_ref[...] += partial

        @pl.when(step == nsteps - 1)
        def _raw_out():
            for blk in range(N_DEV):
                out_ref[pl.ds(blk * m_per, m_per), :] = (
                    acc_ref[:, pl.ds(blk * n_per, n_per)]
                )

    return pl.pallas_call(
        body,
        grid=(nsteps,),
        out_shape=jax.ShapeDtypeStruct((N_DEV * m_per, n_per), jnp.float32),
        in_specs=[
            pl.BlockSpec((m_per, kc), lambda s: (0, s)),
            pl.BlockSpec((m_per, kc), lambda s: (0, s + nsteps)),
            pl.BlockSpec((kc, n), lambda s: (s, 0)),
            pl.BlockSpec((kc, n), lambda s: (s + nsteps, 0)),
        ],
        out_specs=pl.BlockSpec((N_DEV * m_per, n_per), lambda s: (0, 0)),
        scratch_shapes=[
            pltpu.VMEM((m_per, n), jnp.float32),
        ],
        compiler_params=pltpu.CompilerParams(
            dimension_semantics=("arbitrary",),
            vmem_limit_bytes=64 * 1024 * 1024,
        ),
    )(x, x, w_mat, w_mat)


def _kernel_v3(x, w_mat, variant, vmem_mb=64):
    m_per, k = x.shape
    _, n = w_mat.shape
    n_per = n // N_DEV
    kh = k // 2

    def body(x1_ref, x2_ref, w1_ref, w2_ref, out_ref,
             xb1_ref, xb2_ref, y_ref, amax_sm, amax_tx_ref, amax_rx_ref,
             q_tx_ref, q_rx_ref, amax_send_sems, amax_recv_sems,
             data_send_sems, data_recv_sems):
        s = pl.program_id(0)
        my = lax.axis_index("i")

        @pl.when(s == 0)
        def _prologue():
            if variant != "v3mm":
                barrier = pltpu.get_barrier_semaphore()
                for d in range(1, N_DEV):
                    pl.semaphore_signal(
                        barrier, inc=1,
                        device_id=((my + d) % N_DEV,),
                        device_id_type=pl.DeviceIdType.MESH,
                    )
                pl.semaphore_wait(barrier, N_DEV - 1)
            xb1_ref[...] = x1_ref[...].astype(jnp.bfloat16)
            xb2_ref[...] = x2_ref[...].astype(jnp.bfloat16)

        partial = jnp.dot(
            xb1_ref[...], w1_ref[...].astype(jnp.bfloat16),
            preferred_element_type=jnp.float32,
        ) + jnp.dot(
            xb2_ref[...], w2_ref[...].astype(jnp.bfloat16),
            preferred_element_type=jnp.float32,
        )
        y_ref[:, pl.ds(s * n_per, n_per)] = partial
        pmax = jnp.max(jnp.abs(partial))

        @pl.when(s == 0)
        def _amax_init():
            amax_sm[0] = pmax

        @pl.when(s != 0)
        def _amax_acc():
            amax_sm[0] = jnp.maximum(amax_sm[0], pmax)

        @pl.when(s == N_DEV - 1)
        def _epilogue():
            local_amax = amax_sm[0]
            if variant in ("v3mm", "v3sync"):
                scale0 = local_amax * (1.0 / 127.0)
                inv0 = 127.0 / local_amax
                for blk in range(N_DEV):
                    q = jnp.clip(
                        jnp.round(
                            y_ref[:, pl.ds(blk * n_per, n_per)] * inv0
                        ), -127.0, 127.0,
                    )
                    out_ref[pl.ds(blk * m_per, m_per), :] = (
                        q * scale0
                    ).astype(jnp.bfloat16)
                return
            amax_tx_ref[...] = jnp.full((8, 128), local_amax, jnp.float32)
            amax_rdmas = []
            for d in range(1, N_DEV):
                rdma = pltpu.make_async_remote_copy(
                    src_ref=amax_tx_ref,
                    dst_ref=amax_rx_ref.at[d - 1],
                    send_sem=amax_send_sems.at[d - 1],
                    recv_sem=amax_recv_sems.at[d - 1],
                    device_id=((my + d) % N_DEV,),
                    device_id_type=pl.DeviceIdType.MESH,
                )
                rdma.start()
                amax_rdmas.append(rdma)
            for rdma in amax_rdmas:
                rdma.wait()
            g_amax = local_amax
            for d in range(1, N_DEV):
                g_amax = jnp.maximum(g_amax, amax_rx_ref[d - 1, 0, 0])
            scale = g_amax * (1.0 / 127.0)
            inv_scale = 127.0 / g_amax

            data_rdmas = []
            for d in range(1, N_DEV):
                tgt = (my + d) % N_DEV
                blk = y_ref[:, pl.ds(tgt * n_per, n_per)]
                q_tx_ref[d - 1] = jnp.clip(
                    jnp.round(blk * inv_scale), -127.0, 127.0
                ).astype(jnp.int8)
                if variant != "v3noa2a":
                    rdma = pltpu.make_async_remote_copy(
                        src_ref=q_tx_ref.at[d - 1],
                        dst_ref=q_rx_ref.at[d - 1],
                        send_sem=data_send_sems.at[d - 1],
                        recv_sem=data_recv_sems.at[d - 1],
                        device_id=(tgt,),
                        device_id_type=pl.DeviceIdType.MESH,
                    )
                    rdma.start()
                    data_rdmas.append(rdma)

            own = y_ref[:, pl.ds(my * n_per, n_per)]
            q_own = jnp.clip(jnp.round(own * inv_scale), -127.0, 127.0)
            out_ref[pl.ds(my * m_per, m_per), :] = (
                q_own * scale
            ).astype(jnp.bfloat16)

            for d in range(1, N_DEV):
                src = (my - d) % N_DEV
                if variant != "v3noa2a":
                    data_rdmas[d - 1].wait()
                    q_src = q_rx_ref[d - 1]
                else:
                    q_src = q_tx_ref[d - 1]
                out_ref[pl.ds(src * m_per, m_per), :] = (
                    q_src.astype(jnp.float32) * scale
                ).astype(jnp.bfloat16)

    return pl.pallas_call(
        body,
        grid=(N_DEV,),
        out_shape=jax.ShapeDtypeStruct((N_DEV * m_per, n_per), jnp.bfloat16),
        in_specs=[
            pl.BlockSpec((m_per, kh), lambda s: (0, 0)),
            pl.BlockSpec((m_per, kh), lambda s: (0, 1)),
            pl.BlockSpec((kh, n_per), lambda s: (0, s)),
            pl.BlockSpec((kh, n_per), lambda s: (1, s)),
        ],
        out_specs=pl.BlockSpec((N_DEV * m_per, n_per), lambda s: (0, 0)),
        scratch_shapes=[
            pltpu.VMEM((m_per, kh), jnp.bfloat16),
            pltpu.VMEM((m_per, kh), jnp.bfloat16),
            pltpu.VMEM((m_per, n), jnp.float32),
            pltpu.SMEM((1,), jnp.float32),
            pltpu.VMEM((8, 128), jnp.float32),
            pltpu.VMEM((N_DEV - 1, 8, 128), jnp.float32),
            pltpu.VMEM((N_DEV - 1, m_per, n_per), jnp.int8),
            pltpu.VMEM((N_DEV - 1, m_per, n_per), jnp.int8),
            pltpu.SemaphoreType.DMA((N_DEV - 1,)),
            pltpu.SemaphoreType.DMA((N_DEV - 1,)),
            pltpu.SemaphoreType.DMA((N_DEV - 1,)),
            pltpu.SemaphoreType.DMA((N_DEV - 1,)),
        ],
        compiler_params=pltpu.CompilerParams(
            dimension_semantics=("arbitrary",),
            collective_id=None if variant == "v3mm" else 0,
            vmem_limit_bytes=vmem_mb * 1024 * 1024,
        ),
    )(x, x, w_mat, w_mat)


def _kernel_v4(x, w_mat, variant):
    m_per, k = x.shape
    _, n = w_mat.shape
    n_per = n // N_DEV
    kq = k // 4

    no_comm = variant in ("v4mm", "v4conv", "v4f32")
    if variant in ("v4f32", "v5", "v6"):
        xb_shape = pltpu.VMEM((1, 8, 128), jnp.bfloat16)
    else:
        xb_shape = pltpu.VMEM((4, m_per, kq), jnp.bfloat16)

    def body(x1_ref, x2_ref, x3_ref, x4_ref,
             w1_ref, w2_ref, w3_ref, w4_ref, out_ref,
             xb_ref, wb_ref, y_ref, amax_sm, amax_tx_ref, amax_rx_ref,
             q_tx_ref, q_rx_ref, amax_send_sems, amax_recv_sems,
             data_send_sems, data_recv_sems):
        s = pl.program_id(0)
        my = lax.axis_index("i")
        barrier = None if no_comm else pltpu.get_barrier_semaphore()
        x_refs = [x1_ref, x2_ref, x3_ref, x4_ref]
        w_refs = [w1_ref, w2_ref, w3_ref, w4_ref]

        @pl.when(s == 0)
        def _prologue():
            if barrier is not None:
                for d in range(1, N_DEV):
                    pl.semaphore_signal(
                        barrier, inc=1,
                        device_id=((my + d) % N_DEV,),
                        device_id_type=pl.DeviceIdType.MESH,
                    )
            if variant not in ("v4f32", "v5", "v6"):
                for i in range(4):
                    xb_ref[i] = x_refs[i][...].astype(jnp.bfloat16)

        if variant == "v4conv":
            for i in range(4):
                wb_ref[...] = w_refs[i][...].astype(jnp.bfloat16)
            partial = wb_ref[...].astype(jnp.float32)
        elif variant in ("v4f32", "v5", "v6"):
            partial = jnp.dot(
                x_refs[0][...], w_refs[0][...],
                preferred_element_type=jnp.float32,
            )
            for i in range(1, 4):
                partial += jnp.dot(
                    x_refs[i][...], w_refs[i][...],
                    preferred_element_type=jnp.float32,
                )
        else:
            partial = jnp.dot(
                xb_ref[0], w_refs[0][...].astype(jnp.bfloat16),
                preferred_element_type=jnp.float32,
            )
            for i in range(1, 4):
                partial += jnp.dot(
                    xb_ref[i], w_refs[i][...].astype(jnp.bfloat16),
                    preferred_element_type=jnp.float32,
                )
        y_ref[:, pl.ds(s * n_per, n_per)] = partial
        pmax = jnp.max(jnp.abs(partial))

        @pl.when(s == 0)
        def _amax_init():
            amax_sm[0] = pmax

        @pl.when(s != 0)
        def _amax_acc():
            amax_sm[0] = jnp.maximum(amax_sm[0], pmax)

        @pl.when(s == N_DEV - 1)
        def _epilogue():
            if no_comm:
                inv0 = 127.0 / amax_sm[0]
                scale0 = amax_sm[0] * (1.0 / 127.0)
                for blk in range(N_DEV):
                    qb = jnp.clip(
                        jnp.round(
                            y_ref[:, pl.ds(blk * n_per, n_per)] * inv0
                        ), -127.0, 127.0,
                    )
                    out_ref[pl.ds(blk * m_per, m_per), :] = (
                        qb * scale0
                    ).astype(jnp.bfloat16)
                return

            pl.semaphore_wait(barrier, N_DEV - 1)

            local_amax = amax_sm[0]
            amax_tx_ref[...] = jnp.full((8, 128), local_amax, jnp.float32)
            amax_rdmas = []
            for d in range(1, N_DEV):
                rdma = pltpu.make_async_remote_copy(
                    src_ref=amax_tx_ref,
                    dst_ref=amax_rx_ref.at[d - 1],
                    send_sem=amax_send_sems.at[d - 1],
                    recv_sem=amax_recv_sems.at[d - 1],
                    device_id=((my + d) % N_DEV,),
                    device_id_type=pl.DeviceIdType.MESH,
                )
                rdma.start()
                amax_rdmas.append(rdma)
            for rdma in amax_rdmas:
                rdma.wait()
            g_amax = local_amax
            for d in range(1, N_DEV):
                g_amax = jnp.maximum(g_amax, amax_rx_ref[d - 1, 0, 0])
            scale = g_amax * (1.0 / 127.0)
            inv_scale = 127.0 / g_amax

            if variant == "v6":
                mh = m_per // 2
                data_rdmas = {}
                for c in range(2):
                    rows = pl.ds(c * mh, mh)
                    for d in (2, 1, 3):
                        tgt = (my + d) % N_DEV
                        blk = y_ref[rows, pl.ds(tgt * n_per, n_per)]
                        q_tx_ref[d - 1, rows, :] = jnp.clip(
                            jnp.round(blk * inv_scale), -127.0, 127.0
                        ).astype(jnp.int8)
                        rdma = pltpu.make_async_remote_copy(
                            src_ref=q_tx_ref.at[d - 1, rows, :],
                            dst_ref=q_rx_ref.at[d - 1, rows, :],
                            send_sem=data_send_sems.at[d - 1, c],
                            recv_sem=data_recv_sems.at[d - 1, c],
                            device_id=(tgt,),
                            device_id_type=pl.DeviceIdType.MESH,
                        )
                        rdma.start()
                        data_rdmas[(d, c)] = rdma

                own = y_ref[:, pl.ds(my * n_per, n_per)]
                q_own = jnp.clip(jnp.round(own * inv_scale), -127.0, 127.0)
                out_ref[pl.ds(my * m_per, m_per), :] = (
                    q_own * scale
                ).astype(jnp.bfloat16)

                for c in range(2):
                    rows = pl.ds(c * mh, mh)
                    for d in range(1, N_DEV):
                        src = (my - d) % N_DEV
                        data_rdmas[(d, c)].wait()
                        out_ref[pl.ds(src * m_per + c * mh, mh), :] = (
                            q_rx_ref[d - 1, rows, :].astype(jnp.float32)
                            * scale
                        ).astype(jnp.bfloat16)
                return

            data_rdmas = {}
            for d in (2, 1, 3):
                tgt = (my + d) % N_DEV
                blk = y_ref[:, pl.ds(tgt * n_per, n_per)]
                q_tx_ref[d - 1] = jnp.clip(
                    jnp.round(blk * inv_scale), -127.0, 127.0
                ).astype(jnp.int8)
                rdma = pltpu.make_async_remote_copy(
                    src_ref=q_tx_ref.at[d - 1],
                    dst_ref=q_rx_ref.at[d - 1],
                    send_sem=data_send_sems.at[d - 1, 0],
                    recv_sem=data_recv_sems.at[d - 1, 0],
                    device_id=(tgt,),
                    device_id_type=pl.DeviceIdType.MESH,
                )
                rdma.start()
                data_rdmas[d] = rdma

            own = y_ref[:, pl.ds(my * n_per, n_per)]
            q_own = jnp.clip(jnp.round(own * inv_scale), -127.0, 127.0)
            out_ref[pl.ds(my * m_per, m_per), :] = (
                q_own * scale
            ).astype(jnp.bfloat16)

            for d in range(1, N_DEV):
                src = (my - d) % N_DEV
                data_rdmas[d].wait()
                out_ref[pl.ds(src * m_per, m_per), :] = (
                    q_rx_ref[d - 1].astype(jnp.float32) * scale
                ).astype(jnp.bfloat16)

    return pl.pallas_call(
        body,
        grid=(N_DEV,),
        out_shape=jax.ShapeDtypeStruct((N_DEV * m_per, n_per), jnp.bfloat16),
        in_specs=(
            [pl.BlockSpec((m_per, kq), (lambda s, i=i: (0, i)))
             for i in range(4)]
            + [pl.BlockSpec((kq, n_per), (lambda s, i=i: (i, s)))
               for i in range(4)]
        ),
        out_specs=pl.BlockSpec((N_DEV * m_per, n_per), lambda s: (0, 0)),
        scratch_shapes=[
            xb_shape,
            pltpu.VMEM((kq, n_per), jnp.bfloat16),
            pltpu.VMEM((m_per, n), jnp.float32),
            pltpu.SMEM((1,), jnp.float32),
            pltpu.VMEM((8, 128), jnp.float32),
            pltpu.VMEM((N_DEV - 1, 8, 128), jnp.float32),
            pltpu.VMEM((N_DEV - 1, m_per, n_per), jnp.int8),
            pltpu.VMEM((N_DEV - 1, m_per, n_per), jnp.int8),
            pltpu.SemaphoreType.DMA((N_DEV - 1,)),
            pltpu.SemaphoreType.DMA((N_DEV - 1,)),
            pltpu.SemaphoreType.DMA((N_DEV - 1, 2)),
            pltpu.SemaphoreType.DMA((N_DEV - 1, 2)),
        ],
        compiler_params=pltpu.CompilerParams(
            dimension_semantics=("arbitrary",),
            collective_id=None if no_comm else 0,
            vmem_limit_bytes=100 * 1024 * 1024,
        ),
    )(x, x, x, x, w_mat, w_mat, w_mat, w_mat)


def _kernel_v7(x, w_mat, variant):
    m_per, k = x.shape
    _, n = w_mat.shape
    n_per = n // N_DEV
    kq = k // 4

    def w_panel(s):
        if variant == "v8":
            return (lax.axis_index("i") + 1 + s) % N_DEV
        return s

    def body(x1_ref, x2_ref, x3_ref, x4_ref,
             w1_ref, w2_ref, w3_ref, w4_ref, out_ref,
             y_ref, txb_ref, rxb_ref, amax_sm, amax_tx_ref, amax_rx_ref,
             tx_sems, rx_sems, amax_send_sems, amax_recv_sems):
        s = pl.program_id(0)
        my = lax.axis_index("i")
        barrier = pltpu.get_barrier_semaphore()
        x_refs = [x1_ref, x2_ref, x3_ref, x4_ref]
        w_refs = [w1_ref, w2_ref, w3_ref, w4_ref]

        @pl.when(s == 0)
        def _entry_barrier():
            for d in range(1, N_DEV):
                pl.semaphore_signal(
                    barrier, inc=1,
                    device_id=((my + d) % N_DEV,),
                    device_id_type=pl.DeviceIdType.MESH,
                )
            pl.semaphore_wait(barrier, N_DEV - 1)

        p = (my + 1 + s) % N_DEV if variant == "v8" else s
        partial = jnp.dot(
            x_refs[0][...], w_refs[0][...],
            preferred_element_type=jnp.float32,
        )
        for i in range(1, 4):
            partial += jnp.dot(
                x_refs[i][...], w_refs[i][...],
                preferred_element_type=jnp.float32,
            )
        y_ref[:, pl.ds(p * n_per, n_per)] = partial
        pmax = jnp.max(jnp.abs(partial))

        @pl.when(s == 0)
        def _amax_init():
            amax_sm[0] = pmax

        @pl.when(s != 0)
        def _amax_acc():
            amax_sm[0] = jnp.maximum(amax_sm[0], pmax)

        @pl.when(p != my)
        def _send_panel():
            txb_ref[p] = partial.astype(jnp.bfloat16)
            rdma = pltpu.make_async_remote_copy(
                src_ref=txb_ref.at[p],
                dst_ref=rxb_ref.at[my],
                send_sem=tx_sems.at[p],
                recv_sem=rx_sems.at[my],
                device_id=(p,),
                device_id_type=pl.DeviceIdType.MESH,
            )
            rdma.start()

        @pl.when(s == N_DEV - 1)
        def _epilogue():
            local_amax = amax_sm[0]
            amax_tx_ref[...] = jnp.full((8, 128), local_amax, jnp.float32)
            amax_rdmas = []
            for d in range(1, N_DEV):
                rdma = pltpu.make_async_remote_copy(
                    src_ref=amax_tx_ref,
                    dst_ref=amax_rx_ref.at[d - 1],
                    send_sem=amax_send_sems.at[d - 1],
                    recv_sem=amax_recv_sems.at[d - 1],
                    device_id=((my + d) % N_DEV,),
                    device_id_type=pl.DeviceIdType.MESH,
                )
                rdma.start()
                amax_rdmas.append(rdma)
            for rdma in amax_rdmas:
                rdma.wait()
            g_amax = local_amax
            for d in range(1, N_DEV):
                g_amax = jnp.maximum(g_amax, amax_rx_ref[d - 1, 0, 0])
            scale = g_amax * (1.0 / 127.0)
            inv_scale = 127.0 / g_amax

            own = y_ref[:, pl.ds(my * n_per, n_per)]
            q_own = jnp.clip(jnp.round(own * inv_scale), -127.0, 127.0)
            out_ref[pl.ds(my * m_per, m_per), :] = (
                q_own * scale
            ).astype(jnp.bfloat16)

            for d in range(1, N_DEV):
                src = (my - d) % N_DEV
                recv = pltpu.make_async_remote_copy(
                    src_ref=txb_ref.at[0],
                    dst_ref=rxb_ref.at[src],
                    send_sem=tx_sems.at[0],
                    recv_sem=rx_sems.at[src],
                    device_id=(src,),
                    device_id_type=pl.DeviceIdType.MESH,
                )
                recv.wait_recv()
                blk = rxb_ref[src].astype(jnp.float32)
                qb = jnp.clip(jnp.round(blk * inv_scale), -127.0, 127.0)
                out_ref[pl.ds(src * m_per, m_per), :] = (
                    qb * scale
                ).astype(jnp.bfloat16)

            for p in range(N_DEV):
                @pl.when(p != my)
                def _drain(p=p):
                    send = pltpu.make_async_remote_copy(
                        src_ref=txb_ref.at[p],
                        dst_ref=rxb_ref.at[my],
                        send_sem=tx_sems.at[p],
                        recv_sem=rx_sems.at[my],
                        device_id=(p,),
                        device_id_type=pl.DeviceIdType.MESH,
                    )
                    send.wait_send()

    return pl.pallas_call(
        body,
        grid=(N_DEV,),
        out_shape=jax.ShapeDtypeStruct((N_DEV * m_per, n_per), jnp.bfloat16),
        in_specs=(
            [pl.BlockSpec((m_per, kq), (lambda s, i=i: (0, i)))
             for i in range(4)]
            + [pl.BlockSpec((kq, n_per), (lambda s, i=i: (i, w_panel(s))))
               for i in range(4)]
        ),
        out_specs=pl.BlockSpec((N_DEV * m_per, n_per), lambda s: (0, 0)),
        scratch_shapes=[
            pltpu.VMEM((m_per, n), jnp.float32),
            pltpu.VMEM((N_DEV, m_per, n_per), jnp.bfloat16),
            pltpu.VMEM((N_DEV, m_per, n_per), jnp.bfloat16),
            pltpu.SMEM((1,), jnp.float32),
            pltpu.VMEM((8, 128), jnp.float32),
            pltpu.VMEM((N_DEV - 1, 8, 128), jnp.float32),
            pltpu.SemaphoreType.DMA((N_DEV,)),
            pltpu.SemaphoreType.DMA((N_DEV,)),
            pltpu.SemaphoreType.DMA((N_DEV - 1,)),
            pltpu.SemaphoreType.DMA((N_DEV - 1,)),
        ],
        compiler_params=pltpu.CompilerParams(
            dimension_semantics=("arbitrary",),
            collective_id=0,
            vmem_limit_bytes=100 * 1024 * 1024,
        ),
    )(x, x, x, x, w_mat, w_mat, w_mat, w_mat)


def kernel(x, w_mat, variant="v5", kc=KC):
    if variant in ("v7", "v8"):
        return _kernel_v7(x, w_mat, variant)
    m_per, k = x.shape
    _, n = w_mat.shape
    n_per = n // N_DEV
    nsteps = k // kc

    if variant in ("stream4", "mm4"):
        return _kernel4(x, w_mat, variant, kc)
    if variant.startswith("v3"):
        return _kernel_v3(x, w_mat, variant, vmem_mb=(kc if kc > 8 else 64))
    if variant.startswith("v4") or variant in ("v5", "v6"):
        return _kernel_v4(x, w_mat, variant)

    def body(x_ref, w_ref, out_ref, acc_ref, amax_tx_ref, amax_rx_ref,
             q_tx_ref, q_rx_ref, amax_send_sems, amax_recv_sems,
             data_send_sems, data_recv_sems):
        step = pl.program_id(0)
        my = lax.axis_index("i")

        if variant not in ("gemm", "mm"):
            @pl.when(step == 0)
            def _entry_barrier():
                barrier = pltpu.get_barrier_semaphore()
                for d in range(1, N_DEV):
                    pl.semaphore_signal(
                        barrier, inc=1,
                        device_id=((my + d) % N_DEV,),
                        device_id_type=pl.DeviceIdType.MESH,
                    )
                pl.semaphore_wait(barrier, N_DEV - 1)

        if variant == "stream":
            acc_ref[:kc, :] = w_ref[...]
            acc_ref[:m_per, :kc] += x_ref[...]

            @pl.when(step == nsteps - 1)
            def _stream_out():
                out_ref[...] = jnp.concatenate(
                    [acc_ref[:, :n_per]] * N_DEV, axis=0
                )
            return

        partial = jnp.dot(
            x_ref[...].astype(jnp.bfloat16),
            w_ref[...].astype(jnp.bfloat16),
            preferred_element_type=jnp.float32,
        )

        @pl.when(step == 0)
        def _init_acc():
            acc_ref[...] = partial

        @pl.when(step != 0)
        def _accum():
            acc_ref[...] += partial

        if variant == "mm":
            @pl.when(step == nsteps - 1)
            def _raw_out():
                for blk in range(N_DEV):
                    out_ref[pl.ds(blk * m_per, m_per), :] = (
                        acc_ref[:, pl.ds(blk * n_per, n_per)]
                    )
            return

        @pl.when(step == nsteps - 1)
        def _epilogue():
            local_amax = jnp.max(jnp.abs(acc_ref[...]))

            if variant == "gemm":
                g_amax = local_amax
            else:
                amax_tx_ref[...] = jnp.full((8, 128), local_amax, jnp.float32)
                amax_rdmas = []
                for d in range(1, N_DEV):
                    rdma = pltpu.make_async_remote_copy(
                        src_ref=amax_tx_ref,
                        dst_ref=amax_rx_ref.at[d - 1],
                        send_sem=amax_send_sems.at[d - 1],
                        recv_sem=amax_recv_sems.at[d - 1],
                        device_id=((my + d) % N_DEV,),
                        device_id_type=pl.DeviceIdType.MESH,
                    )
                    rdma.start()
                    amax_rdmas.append(rdma)
                for rdma in amax_rdmas:
                    rdma.wait()
                g_amax = local_amax
                for d in range(1, N_DEV):
                    g_amax = jnp.maximum(g_amax, amax_rx_ref[d - 1, 0, 0])

            scale = g_amax * (1.0 / 127.0)
            inv_scale = 127.0 / g_amax

            data_rdmas = []
            for d in range(1, N_DEV):
                tgt = (my + d) % N_DEV
                blk = acc_ref[:, pl.ds(tgt * n_per, n_per)]
                q_tx_ref[d - 1] = jnp.clip(
                    jnp.round(blk * inv_scale), -127.0, 127.0
                ).astype(jnp.int8)
                if variant == "full":
                    rdma = pltpu.make_async_remote_copy(
                        src_ref=q_tx_ref.at[d - 1],
                        dst_ref=q_rx_ref.at[d - 1],
                        send_sem=data_send_sems.at[d - 1],
                        recv_sem=data_recv_sems.at[d - 1],
                        device_id=(tgt,),
                        device_id_type=pl.DeviceIdType.MESH,
                    )
                    rdma.start()
                    data_rdmas.append(rdma)

            own = acc_ref[:, pl.ds(my * n_per, n_per)]
            q_own = jnp.clip(jnp.round(own * inv_scale), -127.0, 127.0)
            out_ref[pl.ds(my * m_per, m_per), :] = q_own * scale

            for d in range(1, N_DEV):
                src = (my - d) % N_DEV
                if variant == "full":
                    data_rdmas[d - 1].wait()
                    q_src = q_rx_ref[d - 1]
                else:
                    q_src = q_tx_ref[d - 1]
                out_ref[pl.ds(src * m_per, m_per), :] = (
                    q_src.astype(jnp.float32) * scale
                )

    return pl.pallas_call(
        body,
        grid=(nsteps,),
        out_shape=jax.ShapeDtypeStruct((N_DEV * m_per, n_per), jnp.float32),
        in_specs=[
            pl.BlockSpec((m_per, kc), lambda s: (0, s)),
            pl.BlockSpec((kc, n), lambda s: (s, 0)),
        ],
        out_specs=pl.BlockSpec((N_DEV * m_per, n_per), lambda s: (0, 0)),
        scratch_shapes=[
            pltpu.VMEM((m_per, n), jnp.float32),
            pltpu.VMEM((8, 128), jnp.float32),
            pltpu.VMEM((N_DEV - 1, 8, 128), jnp.float32),
            pltpu.VMEM((N_DEV - 1, m_per, n_per), jnp.int8),
            pltpu.VMEM((N_DEV - 1, m_per, n_per), jnp.int8),
            pltpu.SemaphoreType.DMA((N_DEV - 1,)),
            pltpu.SemaphoreType.DMA((N_DEV - 1,)),
            pltpu.SemaphoreType.DMA((N_DEV - 1,)),
            pltpu.SemaphoreType.DMA((N_DEV - 1,)),
        ],
        compiler_params=pltpu.CompilerParams(
            dimension_semantics=("arbitrary",),
            collective_id=None if variant in ("gemm", "mm") else 0,
            vmem_limit_bytes=64 * 1024 * 1024,
        ),
    )(x, w_mat)


# device time: 54948 ns/iter; 1.1589x vs baseline; 1.0013x over previous
import jax
import jax.numpy as jnp
from jax import lax
from jax.experimental import pallas as pl
from jax.experimental.pallas import tpu as pltpu

N_DEV = 4
KC = 512


def _kernel4(x, w_mat, variant, kc):
    m_per, k = x.shape
    _, n = w_mat.shape
    n_per = n // N_DEV
    kh = k // 2
    nsteps = kh // kc

    def body(x1_ref, x2_ref, w1_ref, w2_ref, out_ref, acc_ref):
        step = pl.program_id(0)

        if variant == "stream4":
            acc_ref[:kc, :] = w1_ref[...]
            acc_ref[kc:2 * kc, :] = w2_ref[...]
            acc_ref[:m_per, :kc] += x1_ref[...]
            acc_ref[:m_per, kc:2 * kc] += x2_ref[...]

            @pl.when(step == nsteps - 1)
            def _stream_out():
                out_ref[...] = jnp.concatenate(
                    [acc_ref[:m_per, :n_per]] * N_DEV, axis=0
                )
            return

        partial = jnp.dot(
            x1_ref[...].astype(jnp.bfloat16),
            w1_ref[...].astype(jnp.bfloat16),
            preferred_element_type=jnp.float32,
        ) + jnp.dot(
            x2_ref[...].astype(jnp.bfloat16),
            w2_ref[...].astype(jnp.bfloat16),
            preferred_element_type=jnp.float32,
        )

        @pl.when(step == 0)
        def _init_acc():
            acc_ref[...] = partial

        @pl.when(step != 0)
        def _accum():
            acc_ref[...] += partial

        @pl.when(step == nsteps - 1)
        def _raw_out():
            for blk in range(N_DEV):
                out_ref[pl.ds(blk * m_per, m_per), :] = (
                    acc_ref[:, pl.ds(blk * n_per, n_per)]
                )

    return pl.pallas_call(
        body,
        grid=(nsteps,),
        out_shape=jax.ShapeDtypeStruct((N_DEV * m_per, n_per), jnp.float32),
        in_specs=[
            pl.BlockSpec((m_per, kc), lambda s: (0, s)),
            pl.BlockSpec((m_per, kc), lambda s: (0, s + nsteps)),
            pl.BlockSpec((kc, n), lambda s: (s, 0)),
            pl.BlockSpec((kc, n), lambda s: (s + nsteps, 0)),
        ],
        out_specs=pl.BlockSpec((N_DEV * m_per, n_per), lambda s: (0, 0)),
        scratch_shapes=[
            pltpu.VMEM((m_per, n), jnp.float32),
        ],
        compiler_params=pltpu.CompilerParams(
            dimension_semantics=("arbitrary",),
            vmem_limit_bytes=64 * 1024 * 1024,
        ),
    )(x, x, w_mat, w_mat)


def _kernel_v3(x, w_mat, variant, vmem_mb=64):
    m_per, k = x.shape
    _, n = w_mat.shape
    n_per = n // N_DEV
    kh = k // 2

    def body(x1_ref, x2_ref, w1_ref, w2_ref, out_ref,
             xb1_ref, xb2_ref, y_ref, amax_sm, amax_tx_ref, amax_rx_ref,
             q_tx_ref, q_rx_ref, amax_send_sems, amax_recv_sems,
             data_send_sems, data_recv_sems):
        s = pl.program_id(0)
        my = lax.axis_index("i")

        @pl.when(s == 0)
        def _prologue():
            if variant != "v3mm":
                barrier = pltpu.get_barrier_semaphore()
                for d in range(1, N_DEV):
                    pl.semaphore_signal(
                        barrier, inc=1,
                        device_id=((my + d) % N_DEV,),
                        device_id_type=pl.DeviceIdType.MESH,
                    )
                pl.semaphore_wait(barrier, N_DEV - 1)
            xb1_ref[...] = x1_ref[...].astype(jnp.bfloat16)
            xb2_ref[...] = x2_ref[...].astype(jnp.bfloat16)

        partial = jnp.dot(
            xb1_ref[...], w1_ref[...].astype(jnp.bfloat16),
            preferred_element_type=jnp.float32,
        ) + jnp.dot(
            xb2_ref[...], w2_ref[...].astype(jnp.bfloat16),
            preferred_element_type=jnp.float32,
        )
        y_ref[:, pl.ds(s * n_per, n_per)] = partial
        pmax = jnp.max(jnp.abs(partial))

        @pl.when(s == 0)
        def _amax_init():
            amax_sm[0] = pmax

        @pl.when(s != 0)
        def _amax_acc():
            amax_sm[0] = jnp.maximum(amax_sm[0], pmax)

        @pl.when(s == N_DEV - 1)
        def _epilogue():
            local_amax = amax_sm[0]
            if variant in ("v3mm", "v3sync"):
                scale0 = local_amax * (1.0 / 127.0)
                inv0 = 127.0 / local_amax
                for blk in range(N_DEV):
                    q = jnp.clip(
                        jnp.round(
                            y_ref[:, pl.ds(blk * n_per, n_per)] * inv0
                        ), -127.0, 127.0,
                    )
                    out_ref[pl.ds(blk * m_per, m_per), :] = (
                        q * scale0
                    ).astype(jnp.bfloat16)
                return
            amax_tx_ref[...] = jnp.full((8, 128), local_amax, jnp.float32)
            amax_rdmas = []
            for d in range(1, N_DEV):
                rdma = pltpu.make_async_remote_copy(
                    src_ref=amax_tx_ref,
                    dst_ref=amax_rx_ref.at[d - 1],
                    send_sem=amax_send_sems.at[d - 1],
                    recv_sem=amax_recv_sems.at[d - 1],
                    device_id=((my + d) % N_DEV,),
                    device_id_type=pl.DeviceIdType.MESH,
                )
                rdma.start()
                amax_rdmas.append(rdma)
            for rdma in amax_rdmas:
                rdma.wait()
            g_amax = local_amax
            for d in range(1, N_DEV):
                g_amax = jnp.maximum(g_amax, amax_rx_ref[d - 1, 0, 0])
            scale = g_amax * (1.0 / 127.0)
            inv_scale = 127.0 / g_amax

            data_rdmas = []
            for d in range(1, N_DEV):
                tgt = (my + d) % N_DEV
                blk = y_ref[:, pl.ds(tgt * n_per, n_per)]
                q_tx_ref[d - 1] = jnp.clip(
                    jnp.round(blk * inv_scale), -127.0, 127.0
                ).astype(jnp.int8)
                if variant != "v3noa2a":
                    rdma = pltpu.make_async_remote_copy(
                        src_ref=q_tx_ref.at[d - 1],
                        dst_ref=q_rx_ref.at[d - 1],
                        send_sem=data_send_sems.at[d - 1],
                        recv_sem=data_recv_sems.at[d - 1],
                        device_id=(tgt,),
                        device_id_type=pl.DeviceIdType.MESH,
                    )
                    rdma.start()
                    data_rdmas.append(rdma)

            own = y_ref[:, pl.ds(my * n_per, n_per)]
            q_own = jnp.clip(jnp.round(own * inv_scale), -127.0, 127.0)
            out_ref[pl.ds(my * m_per, m_per), :] = (
                q_own * scale
            ).astype(jnp.bfloat16)

            for d in range(1, N_DEV):
                src = (my - d) % N_DEV
                if variant != "v3noa2a":
                    data_rdmas[d - 1].wait()
                    q_src = q_rx_ref[d - 1]
                else:
                    q_src = q_tx_ref[d - 1]
                out_ref[pl.ds(src * m_per, m_per), :] = (
                    q_src.astype(jnp.float32) * scale
                ).astype(jnp.bfloat16)

    return pl.pallas_call(
        body,
        grid=(N_DEV,),
        out_shape=jax.ShapeDtypeStruct((N_DEV * m_per, n_per), jnp.bfloat16),
        in_specs=[
            pl.BlockSpec((m_per, kh), lambda s: (0, 0)),
            pl.BlockSpec((m_per, kh), lambda s: (0, 1)),
            pl.BlockSpec((kh, n_per), lambda s: (0, s)),
            pl.BlockSpec((kh, n_per), lambda s: (1, s)),
        ],
        out_specs=pl.BlockSpec((N_DEV * m_per, n_per), lambda s: (0, 0)),
        scratch_shapes=[
            pltpu.VMEM((m_per, kh), jnp.bfloat16),
            pltpu.VMEM((m_per, kh), jnp.bfloat16),
            pltpu.VMEM((m_per, n), jnp.float32),
            pltpu.SMEM((1,), jnp.float32),
            pltpu.VMEM((8, 128), jnp.float32),
            pltpu.VMEM((N_DEV - 1, 8, 128), jnp.float32),
            pltpu.VMEM((N_DEV - 1, m_per, n_per), jnp.int8),
            pltpu.VMEM((N_DEV - 1, m_per, n_per), jnp.int8),
            pltpu.SemaphoreType.DMA((N_DEV - 1,)),
            pltpu.SemaphoreType.DMA((N_DEV - 1,)),
            pltpu.SemaphoreType.DMA((N_DEV - 1,)),
            pltpu.SemaphoreType.DMA((N_DEV - 1,)),
        ],
        compiler_params=pltpu.CompilerParams(
            dimension_semantics=("arbitrary",),
            collective_id=None if variant == "v3mm" else 0,
            vmem_limit_bytes=vmem_mb * 1024 * 1024,
        ),
    )(x, x, w_mat, w_mat)


def _kernel_v4(x, w_mat, variant):
    m_per, k = x.shape
    _, n = w_mat.shape
    n_per = n // N_DEV
    kq = k // 4

    no_comm = variant in ("v4mm", "v4conv", "v4f32", "v4mmbf2x", "v4f322x")
    if variant in ("v4f32", "v5", "v6"):
        xb_shape = pltpu.VMEM((1, 8, 128), jnp.bfloat16)
    else:
        xb_shape = pltpu.VMEM((4, m_per, kq), jnp.bfloat16)

    def body(x1_ref, x2_ref, x3_ref, x4_ref,
             w1_ref, w2_ref, w3_ref, w4_ref, out_ref,
             xb_ref, wb_ref, y_ref, amax_sm, amax_tx_ref, amax_rx_ref,
             q_tx_ref, q_rx_ref, amax_send_sems, amax_recv_sems,
             data_send_sems, data_recv_sems):
        s = pl.program_id(0)
        my = lax.axis_index("i")
        barrier = None if no_comm else pltpu.get_barrier_semaphore()
        x_refs = [x1_ref, x2_ref, x3_ref, x4_ref]
        w_refs = [w1_ref, w2_ref, w3_ref, w4_ref]

        @pl.when(s == 0)
        def _prologue():
            if barrier is not None:
                for d in range(1, N_DEV):
                    pl.semaphore_signal(
                        barrier, inc=1,
                        device_id=((my + d) % N_DEV,),
                        device_id_type=pl.DeviceIdType.MESH,
                    )
            if variant not in ("v4f32", "v5", "v6"):
                for i in range(4):
                    xb_ref[i] = x_refs[i][...].astype(jnp.bfloat16)

        if variant == "v4conv":
            for i in range(4):
                wb_ref[...] = w_refs[i][...].astype(jnp.bfloat16)
            partial = wb_ref[...].astype(jnp.float32)
        elif variant in ("v4f32", "v5", "v6"):
            partial = jnp.dot(
                x_refs[0][...], w_refs[0][...],
                preferred_element_type=jnp.float32,
            )
            for i in range(1, 4):
                partial += jnp.dot(
                    x_refs[i][...], w_refs[i][...],
                    preferred_element_type=jnp.float32,
                )
        elif variant == "v4f322x":
            partial = jnp.dot(
                x_refs[0][...], w_refs[0][...],
                preferred_element_type=jnp.float32,
            )
            for r in range(2):
                for i in range(4):
                    if r == 0 and i == 0:
                        continue
                    partial += jnp.dot(
                        x_refs[i][...], w_refs[i][...],
                        preferred_element_type=jnp.float32,
                    )
        else:
            reps = 2 if variant == "v4mmbf2x" else 1
            wb_bf = [w_refs[i][...].astype(jnp.bfloat16) for i in range(4)]
            partial = jnp.dot(
                xb_ref[0], wb_bf[0],
                preferred_element_type=jnp.float32,
            )
            for r in range(reps):
                for i in range(4):
                    if r == 0 and i == 0:
                        continue
                    partial += jnp.dot(
                        xb_ref[i], wb_bf[i],
                        preferred_element_type=jnp.float32,
                    )
        y_ref[:, pl.ds(s * n_per, n_per)] = partial
        pmax = jnp.max(jnp.abs(partial))

        @pl.when(s == 0)
        def _amax_init():
            amax_sm[0] = pmax

        @pl.when(s != 0)
        def _amax_acc():
            amax_sm[0] = jnp.maximum(amax_sm[0], pmax)

        @pl.when(s == N_DEV - 1)
        def _epilogue():
            if no_comm:
                inv0 = 127.0 / amax_sm[0]
                scale0 = amax_sm[0] * (1.0 / 127.0)
                for blk in range(N_DEV):
                    qb = jnp.clip(
                        jnp.round(
                            y_ref[:, pl.ds(blk * n_per, n_per)] * inv0
                        ), -127.0, 127.0,
                    )
                    out_ref[pl.ds(blk * m_per, m_per), :] = (
                        qb * scale0
                    ).astype(jnp.bfloat16)
                return

            pl.semaphore_wait(barrier, N_DEV - 1)

            local_amax = amax_sm[0]
            amax_tx_ref[...] = jnp.full((8, 128), local_amax, jnp.float32)
            amax_rdmas = []
            for d in range(1, N_DEV):
                rdma = pltpu.make_async_remote_copy(
                    src_ref=amax_tx_ref,
                    dst_ref=amax_rx_ref.at[d - 1],
                    send_sem=amax_send_sems.at[d - 1],
                    recv_sem=amax_recv_sems.at[d - 1],
                    device_id=((my + d) % N_DEV,),
                    device_id_type=pl.DeviceIdType.MESH,
                )
                rdma.start()
                amax_rdmas.append(rdma)
            for rdma in amax_rdmas:
                rdma.wait()
            g_amax = local_amax
            for d in range(1, N_DEV):
                g_amax = jnp.maximum(g_amax, amax_rx_ref[d - 1, 0, 0])
            scale = g_amax * (1.0 / 127.0)
            inv_scale = 127.0 / g_amax

            if variant == "v6":
                mh = m_per // 2
                data_rdmas = {}
                for c in range(2):
                    rows = pl.ds(c * mh, mh)
                    for d in (2, 1, 3):
                        tgt = (my + d) % N_DEV
                        blk = y_ref[rows, pl.ds(tgt * n_per, n_per)]
                        q_tx_ref[d - 1, rows, :] = jnp.clip(
                            jnp.round(blk * inv_scale), -127.0, 127.0
                        ).astype(jnp.int8)
                        rdma = pltpu.make_async_remote_copy(
                            src_ref=q_tx_ref.at[d - 1, rows, :],
                            dst_ref=q_rx_ref.at[d - 1, rows, :],
                            send_sem=data_send_sems.at[d - 1, c],
                            recv_sem=data_recv_sems.at[d - 1, c],
                            device_id=(tgt,),
                            device_id_type=pl.DeviceIdType.MESH,
                        )
                        rdma.start()
                        data_rdmas[(d, c)] = rdma

                own = y_ref[:, pl.ds(my * n_per, n_per)]
                q_own = jnp.clip(jnp.round(own * inv_scale), -127.0, 127.0)
                out_ref[pl.ds(my * m_per, m_per), :] = (
                    q_own * scale
                ).astype(jnp.bfloat16)

                for c in range(2):
                    rows = pl.ds(c * mh, mh)
                    for d in range(1, N_DEV):
                        src = (my - d) % N_DEV
                        data_rdmas[(d, c)].wait()
                        out_ref[pl.ds(src * m_per + c * mh, mh), :] = (
                            q_rx_ref[d - 1, rows, :].astype(jnp.float32)
                            * scale
                        ).astype(jnp.bfloat16)
                return

            data_rdmas = {}
            for d in (2, 1, 3):
                tgt = (my + d) % N_DEV
                blk = y_ref[:, pl.ds(tgt * n_per, n_per)]
                q_tx_ref[d - 1] = jnp.clip(
                    jnp.round(blk * inv_scale), -127.0, 127.0
                ).astype(jnp.int8)
                rdma = pltpu.make_async_remote_copy(
                    src_ref=q_tx_ref.at[d - 1],
                    dst_ref=q_rx_ref.at[d - 1],
                    send_sem=data_send_sems.at[d - 1, 0],
                    recv_sem=data_recv_sems.at[d - 1, 0],
                    device_id=(tgt,),
                    device_id_type=pl.DeviceIdType.MESH,
                )
                rdma.start()
                data_rdmas[d] = rdma

            own = y_ref[:, pl.ds(my * n_per, n_per)]
            q_own = jnp.clip(jnp.round(own * inv_scale), -127.0, 127.0)
            out_ref[pl.ds(my * m_per, m_per), :] = (
                q_own * scale
            ).astype(jnp.bfloat16)

            for d in range(1, N_DEV):
                src = (my - d) % N_DEV
                data_rdmas[d].wait()
                out_ref[pl.ds(src * m_per, m_per), :] = (
                    q_rx_ref[d - 1].astype(jnp.float32) * scale
                ).astype(jnp.bfloat16)

    return pl.pallas_call(
        body,
        grid=(N_DEV,),
        out_shape=jax.ShapeDtypeStruct((N_DEV * m_per, n_per), jnp.bfloat16),
        in_specs=(
            [pl.BlockSpec((m_per, kq), (lambda s, i=i: (0, i)))
             for i in range(4)]
            + [pl.BlockSpec((kq, n_per), (lambda s, i=i: (i, s)))
               for i in range(4)]
        ),
        out_specs=pl.BlockSpec((N_DEV * m_per, n_per), lambda s: (0, 0)),
        scratch_shapes=[
            xb_shape,
            pltpu.VMEM((kq, n_per), jnp.bfloat16),
            pltpu.VMEM((m_per, n), jnp.float32),
            pltpu.SMEM((1,), jnp.float32),
            pltpu.VMEM((8, 128), jnp.float32),
            pltpu.VMEM((N_DEV - 1, 8, 128), jnp.float32),
            pltpu.VMEM((N_DEV - 1, m_per, n_per), jnp.int8),
            pltpu.VMEM((N_DEV - 1, m_per, n_per), jnp.int8),
            pltpu.SemaphoreType.DMA((N_DEV - 1,)),
            pltpu.SemaphoreType.DMA((N_DEV - 1,)),
            pltpu.SemaphoreType.DMA((N_DEV - 1, 2)),
            pltpu.SemaphoreType.DMA((N_DEV - 1, 2)),
        ],
        compiler_params=pltpu.CompilerParams(
            dimension_semantics=("arbitrary",),
            collective_id=None if no_comm else 0,
            vmem_limit_bytes=100 * 1024 * 1024,
        ),
    )(x, x, x, x, w_mat, w_mat, w_mat, w_mat)


def _kernel_v7(x, w_mat, variant):
    m_per, k = x.shape
    _, n = w_mat.shape
    n_per = n // N_DEV
    kq = k // 4

    def w_panel(s):
        if variant == "v8":
            return (lax.axis_index("i") + 1 + s) % N_DEV
        return s

    def body(x1_ref, x2_ref, x3_ref, x4_ref,
             w1_ref, w2_ref, w3_ref, w4_ref, out_ref,
             y_ref, txb_ref, rxb_ref, amax_sm, amax_tx_ref, amax_rx_ref,
             tx_sems, rx_sems, amax_send_sems, amax_recv_sems):
        s = pl.program_id(0)
        my = lax.axis_index("i")
        barrier = pltpu.get_barrier_semaphore()
        x_refs = [x1_ref, x2_ref, x3_ref, x4_ref]
        w_refs = [w1_ref, w2_ref, w3_ref, w4_ref]

        @pl.when(s == 0)
        def _entry_barrier():
            for d in range(1, N_DEV):
                pl.semaphore_signal(
                    barrier, inc=1,
                    device_id=((my + d) % N_DEV,),
                    device_id_type=pl.DeviceIdType.MESH,
                )
            pl.semaphore_wait(barrier, N_DEV - 1)

        p = (my + 1 + s) % N_DEV if variant == "v8" else s
        partial = jnp.dot(
            x_refs[0][...], w_refs[0][...],
            preferred_element_type=jnp.float32,
        )
        for i in range(1, 4):
            partial += jnp.dot(
                x_refs[i][...], w_refs[i][...],
                preferred_element_type=jnp.float32,
            )
        y_ref[:, pl.ds(p * n_per, n_per)] = partial
        pmax = jnp.max(jnp.abs(partial))

        @pl.when(s == 0)
        def _amax_init():
            amax_sm[0] = pmax

        @pl.when(s != 0)
        def _amax_acc():
            amax_sm[0] = jnp.maximum(amax_sm[0], pmax)

        @pl.when(p != my)
        def _send_panel():
            txb_ref[p] = partial.astype(jnp.bfloat16)
            rdma = pltpu.make_async_remote_copy(
                src_ref=txb_ref.at[p],
                dst_ref=rxb_ref.at[my],
                send_sem=tx_sems.at[p],
                recv_sem=rx_sems.at[my],
                device_id=(p,),
                device_id_type=pl.DeviceIdType.MESH,
            )
            rdma.start()

        @pl.when(s == N_DEV - 1)
        def _epilogue():
            local_amax = amax_sm[0]
            amax_tx_ref[...] = jnp.full((8, 128), local_amax, jnp.float32)
            amax_rdmas = []
            for d in range(1, N_DEV):
                rdma = pltpu.make_async_remote_copy(
                    src_ref=amax_tx_ref,
                    dst_ref=amax_rx_ref.at[d - 1],
                    send_sem=amax_send_sems.at[d - 1],
                    recv_sem=amax_recv_sems.at[d - 1],
                    device_id=((my + d) % N_DEV,),
                    device_id_type=pl.DeviceIdType.MESH,
                )
                rdma.start()
                amax_rdmas.append(rdma)
            for rdma in amax_rdmas:
                rdma.wait()
            g_amax = local_amax
            for d in range(1, N_DEV):
                g_amax = jnp.maximum(g_amax, amax_rx_ref[d - 1, 0, 0])
            scale = g_amax * (1.0 / 127.0)
            inv_scale = 127.0 / g_amax

            own = y_ref[:, pl.ds(my * n_per, n_per)]
            q_own = jnp.clip(jnp.round(own * inv_scale), -127.0, 127.0)
            out_ref[pl.ds(my * m_per, m_per), :] = (
                q_own * scale
            ).astype(jnp.bfloat16)

            for d in range(1, N_DEV):
                src = (my - d) % N_DEV
                recv = pltpu.make_async_remote_copy(
                    src_ref=txb_ref.at[0],
                    dst_ref=rxb_ref.at[src],
                    send_sem=tx_sems.at[0],
                    recv_sem=rx_sems.at[src],
                    device_id=(src,),
                    device_id_type=pl.DeviceIdType.MESH,
                )
                recv.wait_recv()
                blk = rxb_ref[src].astype(jnp.float32)
                qb = jnp.clip(jnp.round(blk * inv_scale), -127.0, 127.0)
                out_ref[pl.ds(src * m_per, m_per), :] = (
                    qb * scale
                ).astype(jnp.bfloat16)

            for p in range(N_DEV):
                @pl.when(p != my)
                def _drain(p=p):
                    send = pltpu.make_async_remote_copy(
                        src_ref=txb_ref.at[p],
                        dst_ref=rxb_ref.at[my],
                        send_sem=tx_sems.at[p],
                        recv_sem=rx_sems.at[my],
                        device_id=(p,),
                        device_id_type=pl.DeviceIdType.MESH,
                    )
                    send.wait_send()

    return pl.pallas_call(
        body,
        grid=(N_DEV,),
        out_shape=jax.ShapeDtypeStruct((N_DEV * m_per, n_per), jnp.bfloat16),
        in_specs=(
            [pl.BlockSpec((m_per, kq), (lambda s, i=i: (0, i)))
             for i in range(4)]
            + [pl.BlockSpec((kq, n_per), (lambda s, i=i: (i, w_panel(s))))
               for i in range(4)]
        ),
        out_specs=pl.BlockSpec((N_DEV * m_per, n_per), lambda s: (0, 0)),
        scratch_shapes=[
            pltpu.VMEM((m_per, n), jnp.float32),
            pltpu.VMEM((N_DEV, m_per, n_per), jnp.bfloat16),
            pltpu.VMEM((N_DEV, m_per, n_per), jnp.bfloat16),
            pltpu.SMEM((1,), jnp.float32),
            pltpu.VMEM((8, 128), jnp.float32),
            pltpu.VMEM((N_DEV - 1, 8, 128), jnp.float32),
            pltpu.SemaphoreType.DMA((N_DEV,)),
            pltpu.SemaphoreType.DMA((N_DEV,)),
            pltpu.SemaphoreType.DMA((N_DEV - 1,)),
            pltpu.SemaphoreType.DMA((N_DEV - 1,)),
        ],
        compiler_params=pltpu.CompilerParams(
            dimension_semantics=("arbitrary",),
            collective_id=0,
            vmem_limit_bytes=100 * 1024 * 1024,
        ),
    )(x, x, x, x, w_mat, w_mat, w_mat, w_mat)


def kernel(x, w_mat, variant="v5", kc=KC):
    if variant in ("v7", "v8"):
        return _kernel_v7(x, w_mat, variant)
    m_per, k = x.shape
    _, n = w_mat.shape
    n_per = n // N_DEV
    nsteps = k // kc

    if variant in ("stream4", "mm4"):
        return _kernel4(x, w_mat, variant, kc)
    if variant.startswith("v3"):
        return _kernel_v3(x, w_mat, variant, vmem_mb=(kc if kc > 8 else 64))
    if variant.startswith("v4") or variant in ("v5", "v6"):
        return _kernel_v4(x, w_mat, variant)

    def body(x_ref, w_ref, out_ref, acc_ref, amax_tx_ref, amax_rx_ref,
             q_tx_ref, q_rx_ref, amax_send_sems, amax_recv_sems,
             data_send_sems, data_recv_sems):
        step = pl.program_id(0)
        my = lax.axis_index("i")

        if variant not in ("gemm", "mm"):
            @pl.when(step == 0)
            def _entry_barrier():
                barrier = pltpu.get_barrier_semaphore()
                for d in range(1, N_DEV):
                    pl.semaphore_signal(
                        barrier, inc=1,
                        device_id=((my + d) % N_DEV,),
                        device_id_type=pl.DeviceIdType.MESH,
                    )
                pl.semaphore_wait(barrier, N_DEV - 1)

        if variant == "stream":
            acc_ref[:kc, :] = w_ref[...]
            acc_ref[:m_per, :kc] += x_ref[...]

            @pl.when(step == nsteps - 1)
            def _stream_out():
                out_ref[...] = jnp.concatenate(
                    [acc_ref[:, :n_per]] * N_DEV, axis=0
                )
            return

        partial = jnp.dot(
            x_ref[...].astype(jnp.bfloat16),
            w_ref[...].astype(jnp.bfloat16),
            preferred_element_type=jnp.float32,
        )

        @pl.when(step == 0)
        def _init_acc():
            acc_ref[...] = partial

        @pl.when(step != 0)
        def _accum():
            acc_ref[...] += partial

        if variant == "mm":
            @pl.when(step == nsteps - 1)
            def _raw_out():
                for blk in range(N_DEV):
                    out_ref[pl.ds(blk * m_per, m_per), :] = (
                        acc_ref[:, pl.ds(blk * n_per, n_per)]
                    )
            return

        @pl.when(step == nsteps - 1)
        def _epilogue():
            local_amax = jnp.max(jnp.abs(acc_ref[...]))

            if variant == "gemm":
                g_amax = local_amax
            else:
                amax_tx_ref[...] = jnp.full((8, 128), local_amax, jnp.float32)
                amax_rdmas = []
                for d in range(1, N_DEV):
                    rdma = pltpu.make_async_remote_copy(
                        src_ref=amax_tx_ref,
                        dst_ref=amax_rx_ref.at[d - 1],
                        send_sem=amax_send_sems.at[d - 1],
                        recv_sem=amax_recv_sems.at[d - 1],
                        device_id=((my + d) % N_DEV,),
                        device_id_type=pl.DeviceIdType.MESH,
                    )
                    rdma.start()
                    amax_rdmas.append(rdma)
                for rdma in amax_rdmas:
                    rdma.wait()
                g_amax = local_amax
                for d in range(1, N_DEV):
                    g_amax = jnp.maximum(g_amax, amax_rx_ref[d - 1, 0, 0])

            scale = g_amax * (1.0 / 127.0)
            inv_scale = 127.0 / g_amax

            data_rdmas = []
            for d in range(1, N_DEV):
                tgt = (my + d) % N_DEV
                blk = acc_ref[:, pl.ds(tgt * n_per, n_per)]
                q_tx_ref[d - 1] = jnp.clip(
                    jnp.round(blk * inv_scale), -127.0, 127.0
                ).astype(jnp.int8)
                if variant == "full":
                    rdma = pltpu.make_async_remote_copy(
                        src_ref=q_tx_ref.at[d - 1],
                        dst_ref=q_rx_ref.at[d - 1],
                        send_sem=data_send_sems.at[d - 1],
                        recv_sem=data_recv_sems.at[d - 1],
                        device_id=(tgt,),
                        device_id_type=pl.DeviceIdType.MESH,
                    )
                    rdma.start()
                    data_rdmas.append(rdma)

            own = acc_ref[:, pl.ds(my * n_per, n_per)]
            q_own = jnp.clip(jnp.round(own * inv_scale), -127.0, 127.0)
            out_ref[pl.ds(my * m_per, m_per), :] = q_own * scale

            for d in range(1, N_DEV):
                src = (my - d) % N_DEV
                if variant == "full":
                    data_rdmas[d - 1].wait()
                    q_src = q_rx_ref[d - 1]
                else:
                    q_src = q_tx_ref[d - 1]
                out_ref[pl.ds(src * m_per, m_per), :] = (
                    q_src.astype(jnp.float32) * scale
                )

    return pl.pallas_call(
        body,
        grid=(nsteps,),
        out_shape=jax.ShapeDtypeStruct((N_DEV * m_per, n_per), jnp.float32),
        in_specs=[
            pl.BlockSpec((m_per, kc), lambda s: (0, s)),
            pl.BlockSpec((kc, n), lambda s: (s, 0)),
        ],
        out_specs=pl.BlockSpec((N_DEV * m_per, n_per), lambda s: (0, 0)),
        scratch_shapes=[
            pltpu.VMEM((m_per, n), jnp.float32),
            pltpu.VMEM((8, 128), jnp.float32),
            pltpu.VMEM((N_DEV - 1, 8, 128), jnp.float32),
            pltpu.VMEM((N_DEV - 1, m_per, n_per), jnp.int8),
            pltpu.VMEM((N_DEV - 1, m_per, n_per), jnp.int8),
            pltpu.SemaphoreType.DMA((N_DEV - 1,)),
            pltpu.SemaphoreType.DMA((N_DEV - 1,)),
            pltpu.SemaphoreType.DMA((N_DEV - 1,)),
            pltpu.SemaphoreType.DMA((N_DEV - 1,)),
        ],
        compiler_params=pltpu.CompilerParams(
            dimension_semantics=("arbitrary",),
            collective_id=None if variant in ("gemm", "mm") else 0,
            vmem_limit_bytes=64 * 1024 * 1024,
        ),
    )(x, w_mat)
